# baseline (speedup 1.0000x reference)
import functools

import jax
import jax.numpy as jnp
from jax import lax
from jax.experimental import pallas as pl
from jax.experimental.pallas import tpu as pltpu
from jax.experimental.pallas import tpu_sc as plsc

B, S, D = 1, 2048, 1024
H, KVH, HD = 16, 4, 64
E, TOPK, F = 8, 2, 768
EPS = 1e-06

BT = 256
NB = S // BT

NP = TOPK * S
BTM = 128
P = NP + E * BTM
NG = P // BTM
DI = D // 2

_dot = functools.partial(jax.lax.dot_general, preferred_element_type=jnp.float32)

_SC_NC, _SC_NS = 2, 16
_SC_NW = _SC_NC * _SC_NS


def _qkv_body(h_ref, lnw_ref, wqkv_ref, cos_ref, sin_ref, qnw_ref, knw_ref,
              q_ref, k_ref, v_ref):
    h32 = h_ref[...]
    var = jnp.mean(h32 * h32, axis=1, keepdims=True)
    hn = (h32 * jax.lax.rsqrt(var + EPS)) * lnw_ref[...]
    qkv = _dot(hn.astype(jnp.bfloat16), wqkv_ref[...], (((1,), (1,)), ((), ())))
    cos = cos_ref[...]
    sin = sin_ref[...]

    def headnorm_rope(x, w):
        ms = jnp.mean(x * x, axis=1, keepdims=True)
        xn = (x * jax.lax.rsqrt(ms + EPS)) * w
        xr = jnp.concatenate([-xn[:, HD // 2:], xn[:, :HD // 2]], axis=1)
        return xn * cos + xr * sin

    for hh in range(H):
        q = headnorm_rope(qkv[:, hh * HD:(hh + 1) * HD], qnw_ref[...])
        q_ref[hh] = (q * (HD ** -0.5)).astype(jnp.bfloat16)
    for hh in range(KVH):
        k = headnorm_rope(qkv[:, H * HD + hh * HD: H * HD + (hh + 1) * HD],
                          knw_ref[...])
        k_ref[hh] = k.astype(jnp.bfloat16)
    for hh in range(KVH):
        base = (H + KVH) * HD + hh * HD
        v_ref[hh] = qkv[:, base:base + HD].astype(jnp.bfloat16)


def _attn_body(q_ref, k_ref, v_ref, o_ref):
    s = _dot(q_ref[0], k_ref[0], (((1,), (1,)), ((), ())))
    m = jnp.max(s, axis=1, keepdims=True)
    p = jnp.exp(s - m)
    l = jnp.sum(p, axis=1, keepdims=True)
    o = _dot(p.astype(jnp.bfloat16), v_ref[0], (((1,), (0,)), ((), ())))
    o_ref[0] = (o / l).astype(jnp.bfloat16)


def _post_body(attn_ref, wo_ref, res_ref, plw_ref, gate_ref,
               h_ref, x_ref, lg_ref):
    acc = _dot(attn_ref[0], wo_ref[:, 0, :], (((1,), (1,)), ((), ())))
    for hh in range(1, H):
        acc = acc + _dot(attn_ref[hh], wo_ref[:, hh, :], (((1,), (1,)), ((), ())))
    hout = res_ref[...] + acc
    var = jnp.mean(hout * hout, axis=1, keepdims=True)
    xn = (hout * jax.lax.rsqrt(var + EPS)) * plw_ref[...]
    xb = xn.astype(jnp.bfloat16)
    h_ref[...] = hout
    x_ref[...] = xb
    lg_ref[...] = _dot(xb, gate_ref[...], (((1,), (1,)), ((), ())))


def _route_body(lg_ref, w0_ref, w1_ref, pos_ref, nbk_ref):
    lg = lg_ref[...]
    m = jnp.max(lg, axis=1, keepdims=True)
    p = jnp.exp(lg - m)
    rw = p / jnp.sum(p, axis=1, keepdims=True)
    lanes = jax.lax.broadcasted_iota(jnp.int32, (S, E), 1)
    m0 = jnp.max(rw, axis=1, keepdims=True)
    i0 = jnp.min(jnp.where(rw == m0, lanes, E), axis=1, keepdims=True)
    sel0 = lanes == i0
    rw2 = jnp.where(sel0, -1.0, rw)
    m1 = jnp.max(rw2, axis=1, keepdims=True)
    i1 = jnp.min(jnp.where(rw2 == m1, lanes, E), axis=1, keepdims=True)
    sel1 = lanes == i1
    wsum = m0 + m1
    w0_ref[...] = m0 / wsum
    w1_ref[...] = m1 / wsum

    oh = jnp.concatenate([sel0.astype(jnp.float32), sel1.astype(jnp.float32)],
                         axis=0)
    CH = 512
    r_iota = jax.lax.broadcasted_iota(jnp.int32, (CH, CH), 0)
    c_iota = jax.lax.broadcasted_iota(jnp.int32, (CH, CH), 1)
    tri = (c_iota < r_iota).astype(jnp.float32)
    carry = jnp.zeros((1, E), jnp.float32)
    ranks = []
    for c in range(NP // CH):
        ohc = oh[c * CH:(c + 1) * CH]
        ranks.append(_dot(tri, ohc, (((1,), (0,)), ((), ()))) + carry)
        carry = carry + jnp.sum(ohc, axis=0, keepdims=True)
    rank = jnp.concatenate(ranks, axis=0)
    nbk = jnp.floor((carry + (BTM - 1)) * (1.0 / BTM))
    er_iota = jax.lax.broadcasted_iota(jnp.int32, (E, E), 0)
    ec_iota = jax.lax.broadcasted_iota(jnp.int32, (E, E), 1)
    tri_e = (er_iota < ec_iota).astype(jnp.float32)
    off = BTM * _dot(nbk, tri_e, (((1,), (0,)), ((), ())))
    pos_f = (jnp.sum(rank * oh, axis=1, keepdims=True)
             + jnp.sum(off * oh, axis=1, keepdims=True))
    pos_ref[...] = pos_f.astype(jnp.int32)
    nbk_ref[...] = nbk.astype(jnp.int32)


def _sc_dispatch_body(pos_hbm, xi_hbm, xs_hbm, idx_v, rows_v, sem):
    wid = lax.axis_index("s") * _SC_NC + lax.axis_index("c")
    pairs_per_w = NP // _SC_NW
    tok_shift = jnp.where(wid < _SC_NW // 2, 0, S)
    for s_ in range(pairs_per_w // 32):
        j0 = wid * pairs_per_w + s_ * 32
        pltpu.async_copy(xi_hbm.at[pl.ds(j0 - tok_shift, 32)], rows_v,
                         sem).wait()
        pltpu.async_copy(pos_hbm.at[pl.ds(j0, 32)], idx_v, sem).wait()
        pltpu.async_copy(rows_v, xs_hbm.at[idx_v], sem).wait()


def _sc_combine_body(pos_hbm, oi_hbm, a0_hbm, a1_hbm, idx_v, rows_v, sem):
    wid = lax.axis_index("s") * _SC_NC + lax.axis_index("c")
    tok_per_w = S // _SC_NW
    for part in range(2):
        out_hbm = a0_hbm if part == 0 else a1_hbm
        for s_ in range(tok_per_w // 32):
            off = wid * tok_per_w + s_ * 32
            pltpu.async_copy(pos_hbm.at[pl.ds(part * S + off, 32)], idx_v,
                             sem).wait()
            pltpu.async_copy(oi_hbm.at[idx_v], rows_v, sem).wait()
            pltpu.async_copy(rows_v, out_hbm.at[pl.ds(off, 32)], sem).wait()


def _ffn_body(eid_ref, x_ref, wg_ref, wu_ref, wd_ref, o_ref):
    x = x_ref[...]
    g = _dot(x, wg_ref[0], (((1,), (1,)), ((), ())))
    u = _dot(x, wu_ref[0], (((1,), (1,)), ((), ())))
    hexp = ((g * jax.nn.sigmoid(g)) * u).astype(jnp.bfloat16)
    o_ref[...] = _dot(hexp, wd_ref[0], (((1,), (1,)), ((), ()))).astype(
        jnp.bfloat16)


def _combine_body(res_ref, a0_ref, a1_ref, w0_ref, w1_ref, o_ref):
    o_ref[...] = (res_ref[...]
                  + w0_ref[...] * a0_ref[...].astype(jnp.float32)
                  + w1_ref[...] * a1_ref[...].astype(jnp.float32))


def kernel(hidden_states, start_pos, position_cos, position_sin, attention_mask,
           Wq, Wk, Wv, Wo, q_norm_w, k_norm_w, input_ln_w, post_ln_w,
           gate_w, Wg, Wu, Wd):
    x2d = hidden_states.reshape(S, D)
    wqkv = jnp.concatenate([Wq, Wk, Wv], axis=0).astype(jnp.bfloat16)
    wo3 = Wo.reshape(D, H, HD).astype(jnp.bfloat16)
    gate_b = gate_w.astype(jnp.bfloat16)
    wg_b = Wg.astype(jnp.bfloat16)
    wu_b = Wu.astype(jnp.bfloat16)
    wd_b = Wd.astype(jnp.bfloat16)
    lnw = input_ln_w.reshape(1, D)
    plw = post_ln_w.reshape(1, D)
    qnw = q_norm_w.reshape(1, HD)
    knw = k_norm_w.reshape(1, HD)

    q3, k3, v3 = pl.pallas_call(
        _qkv_body,
        grid=(NB,),
        in_specs=[
            pl.BlockSpec((BT, D), lambda i: (i, 0)),
            pl.BlockSpec((1, D), lambda i: (0, 0)),
            pl.BlockSpec(((H + 2 * KVH) * HD, D), lambda i: (0, 0)),
            pl.BlockSpec((BT, HD), lambda i: (i, 0)),
            pl.BlockSpec((BT, HD), lambda i: (i, 0)),
            pl.BlockSpec((1, HD), lambda i: (0, 0)),
            pl.BlockSpec((1, HD), lambda i: (0, 0)),
        ],
        out_specs=[
            pl.BlockSpec((H, BT, HD), lambda i: (0, i, 0)),
            pl.BlockSpec((KVH, BT, HD), lambda i: (0, i, 0)),
            pl.BlockSpec((KVH, BT, HD), lambda i: (0, i, 0)),
        ],
        out_shape=[
            jax.ShapeDtypeStruct((H, S, HD), jnp.bfloat16),
            jax.ShapeDtypeStruct((KVH, S, HD), jnp.bfloat16),
            jax.ShapeDtypeStruct((KVH, S, HD), jnp.bfloat16),
        ],
    )(x2d, lnw, wqkv, position_cos, position_sin, qnw, knw)

    attn3 = pl.pallas_call(
        _attn_body,
        grid=(H, NB),
        in_specs=[
            pl.BlockSpec((1, BT, HD), lambda h, i: (h, i, 0)),
            pl.BlockSpec((1, S, HD), lambda h, i: (h // (H // KVH), 0, 0)),
            pl.BlockSpec((1, S, HD), lambda h, i: (h // (H // KVH), 0, 0)),
        ],
        out_specs=pl.BlockSpec((1, BT, HD), lambda h, i: (h, i, 0)),
        out_shape=jax.ShapeDtypeStruct((H, S, HD), jnp.bfloat16),
    )(q3, k3, v3)

    hres, xb, logits = pl.pallas_call(
        _post_body,
        grid=(NB,),
        in_specs=[
            pl.BlockSpec((H, BT, HD), lambda i: (0, i, 0)),
            pl.BlockSpec((D, H, HD), lambda i: (0, 0, 0)),
            pl.BlockSpec((BT, D), lambda i: (i, 0)),
            pl.BlockSpec((1, D), lambda i: (0, 0)),
            pl.BlockSpec((E, D), lambda i: (0, 0)),
        ],
        out_specs=[
            pl.BlockSpec((BT, D), lambda i: (i, 0)),
            pl.BlockSpec((BT, D), lambda i: (i, 0)),
            pl.BlockSpec((BT, E), lambda i: (i, 0)),
        ],
        out_shape=[
            jax.ShapeDtypeStruct((S, D), jnp.float32),
            jax.ShapeDtypeStruct((S, D), jnp.bfloat16),
            jax.ShapeDtypeStruct((S, E), jnp.float32),
        ],
    )(attn3, wo3, x2d, plw, gate_b)

    w0, w1, pos, nbk = pl.pallas_call(
        _route_body,
        grid=(1,),
        in_specs=[pl.BlockSpec((S, E), lambda i: (0, 0))],
        out_specs=[
            pl.BlockSpec((S, 1), lambda i: (0, 0)),
            pl.BlockSpec((S, 1), lambda i: (0, 0)),
            pl.BlockSpec((NP, 1), lambda i: (0, 0)),
            pl.BlockSpec((1, E), lambda i: (0, 0)),
        ],
        out_shape=[
            jax.ShapeDtypeStruct((S, 1), jnp.float32),
            jax.ShapeDtypeStruct((S, 1), jnp.float32),
            jax.ShapeDtypeStruct((NP, 1), jnp.int32),
            jax.ShapeDtypeStruct((1, E), jnp.int32),
        ],
    )(logits)

    pos1d = pos.reshape(NP)
    cnb = jnp.cumsum(nbk[0])
    eid = jnp.minimum(
        jnp.sum(cnb[:, None] <= jnp.arange(NG)[None, :], axis=0),
        E - 1).astype(jnp.int32)

    mesh = plsc.VectorSubcoreMesh(core_axis_name="c", subcore_axis_name="s")

    xi = jax.lax.bitcast_convert_type(xb.reshape(S, DI, 2), jnp.int32)
    xsi = pl.kernel(
        _sc_dispatch_body,
        mesh=mesh,
        out_type=jax.ShapeDtypeStruct((P, DI), jnp.int32),
        scratch_types=[
            pltpu.VMEM((32,), jnp.int32),
            pltpu.VMEM((32, DI), jnp.int32),
            pltpu.SemaphoreType.DMA,
        ],
    )(pos1d, xi)
    xs = jax.lax.bitcast_convert_type(xsi, jnp.bfloat16).reshape(P, D)

    osorted = pl.pallas_call(
        _ffn_body,
        grid_spec=pltpu.PrefetchScalarGridSpec(
            num_scalar_prefetch=1,
            grid=(NG,),
            in_specs=[
                pl.BlockSpec((BTM, D), lambda i, eid_ref: (i, 0)),
                pl.BlockSpec((1, F, D), lambda i, eid_ref: (eid_ref[i], 0, 0)),
                pl.BlockSpec((1, F, D), lambda i, eid_ref: (eid_ref[i], 0, 0)),
                pl.BlockSpec((1, D, F), lambda i, eid_ref: (eid_ref[i], 0, 0)),
            ],
            out_specs=pl.BlockSpec((BTM, D), lambda i, eid_ref: (i, 0)),
        ),
        out_shape=jax.ShapeDtypeStruct((P, D), jnp.bfloat16),
    )(eid, xs, wg_b, wu_b, wd_b)

    oi = jax.lax.bitcast_convert_type(osorted.reshape(P, DI, 2), jnp.int32)
    a0i, a1i = pl.kernel(
        _sc_combine_body,
        mesh=mesh,
        out_type=[
            jax.ShapeDtypeStruct((S, DI), jnp.int32),
            jax.ShapeDtypeStruct((S, DI), jnp.int32),
        ],
        scratch_types=[
            pltpu.VMEM((32,), jnp.int32),
            pltpu.VMEM((32, DI), jnp.int32),
            pltpu.SemaphoreType.DMA,
        ],
    )(pos1d, oi)
    a0 = jax.lax.bitcast_convert_type(a0i, jnp.bfloat16).reshape(S, D)
    a1 = jax.lax.bitcast_convert_type(a1i, jnp.bfloat16).reshape(S, D)

    out = pl.pallas_call(
        _combine_body,
        grid=(NB,),
        in_specs=[
            pl.BlockSpec((BT, D), lambda i: (i, 0)),
            pl.BlockSpec((BT, D), lambda i: (i, 0)),
            pl.BlockSpec((BT, D), lambda i: (i, 0)),
            pl.BlockSpec((BT, 1), lambda i: (i, 0)),
            pl.BlockSpec((BT, 1), lambda i: (i, 0)),
        ],
        out_specs=pl.BlockSpec((BT, D), lambda i: (i, 0)),
        out_shape=jax.ShapeDtypeStruct((S, D), jnp.float32),
    )(hres, a0, a1, w0, w1)

    return out.reshape(B, S, D)


# speedup vs baseline: 1.7376x; 1.7376x over previous
import functools

import jax
import jax.numpy as jnp
from jax import lax
from jax.experimental import pallas as pl
from jax.experimental.pallas import tpu as pltpu
from jax.experimental.pallas import tpu_sc as plsc

B, S, D = 1, 2048, 1024
H, KVH, HD = 16, 4, 64
E, TOPK, F = 8, 2, 768
EPS = 1e-06

BT = 256
NB = S // BT

NP = TOPK * S
BTM = 128
P = NP + E * BTM
NG = P // BTM

_dot = functools.partial(jax.lax.dot_general, preferred_element_type=jnp.float32)

_SC_NC, _SC_NS = 2, 16
_SC_NW = _SC_NC * _SC_NS


def _qkv_body(h_ref, lnw_ref, wqkv_ref, cos_ref, sin_ref, qnw_ref, knw_ref,
              q_ref, k_ref, v_ref):
    h32 = h_ref[...]
    var = jnp.mean(h32 * h32, axis=1, keepdims=True)
    hn = (h32 * jax.lax.rsqrt(var + EPS)) * lnw_ref[...]
    qkv = _dot(hn.astype(jnp.bfloat16), wqkv_ref[...], (((1,), (1,)), ((), ())))
    cos = cos_ref[...]
    sin = sin_ref[...]

    def headnorm_rope(x, w):
        ms = jnp.mean(x * x, axis=1, keepdims=True)
        xn = (x * jax.lax.rsqrt(ms + EPS)) * w
        xr = jnp.concatenate([-xn[:, HD // 2:], xn[:, :HD // 2]], axis=1)
        return xn * cos + xr * sin

    for hh in range(H):
        q = headnorm_rope(qkv[:, hh * HD:(hh + 1) * HD], qnw_ref[...])
        q_ref[hh] = (q * (HD ** -0.5)).astype(jnp.bfloat16)
    for hh in range(KVH):
        k = headnorm_rope(qkv[:, H * HD + hh * HD: H * HD + (hh + 1) * HD],
                          knw_ref[...])
        k_ref[hh] = k.astype(jnp.bfloat16)
    for hh in range(KVH):
        base = (H + KVH) * HD + hh * HD
        v_ref[hh] = qkv[:, base:base + HD].astype(jnp.bfloat16)


def _attn_body(q_ref, k_ref, v_ref, o_ref):
    s = _dot(q_ref[0], k_ref[0], (((1,), (1,)), ((), ())))
    m = jnp.max(s, axis=1, keepdims=True)
    p = jnp.exp(s - m)
    l = jnp.sum(p, axis=1, keepdims=True)
    o = _dot(p.astype(jnp.bfloat16), v_ref[0], (((1,), (0,)), ((), ())))
    o_ref[0] = (o / l).astype(jnp.bfloat16)


def _post_body(attn_ref, wo_ref, res_ref, plw_ref, gate_ref,
               h_ref, x_ref, lg_ref):
    acc = _dot(attn_ref[0], wo_ref[:, 0, :], (((1,), (1,)), ((), ())))
    for hh in range(1, H):
        acc = acc + _dot(attn_ref[hh], wo_ref[:, hh, :], (((1,), (1,)), ((), ())))
    hout = res_ref[...] + acc
    var = jnp.mean(hout * hout, axis=1, keepdims=True)
    xn = (hout * jax.lax.rsqrt(var + EPS)) * plw_ref[...]
    h_ref[...] = hout
    x_ref[...] = xn
    lg_ref[...] = _dot(xn.astype(jnp.bfloat16), gate_ref[...],
                       (((1,), (1,)), ((), ())))


def _route_body(lg_ref, w0_ref, w1_ref, pos_ref, nbk_ref):
    lg = lg_ref[...]
    m = jnp.max(lg, axis=1, keepdims=True)
    p = jnp.exp(lg - m)
    rw = p / jnp.sum(p, axis=1, keepdims=True)
    lanes = jax.lax.broadcasted_iota(jnp.int32, (S, E), 1)
    m0 = jnp.max(rw, axis=1, keepdims=True)
    i0 = jnp.min(jnp.where(rw == m0, lanes, E), axis=1, keepdims=True)
    sel0 = lanes == i0
    rw2 = jnp.where(sel0, -1.0, rw)
    m1 = jnp.max(rw2, axis=1, keepdims=True)
    i1 = jnp.min(jnp.where(rw2 == m1, lanes, E), axis=1, keepdims=True)
    sel1 = lanes == i1
    wsum = m0 + m1
    w0_ref[...] = m0 / wsum
    w1_ref[...] = m1 / wsum

    oh = jnp.concatenate([sel0.astype(jnp.float32), sel1.astype(jnp.float32)],
                         axis=0)
    CH = 512
    r_iota = jax.lax.broadcasted_iota(jnp.int32, (CH, CH), 0)
    c_iota = jax.lax.broadcasted_iota(jnp.int32, (CH, CH), 1)
    tri = (c_iota < r_iota).astype(jnp.float32)
    carry = jnp.zeros((1, E), jnp.float32)
    ranks = []
    for c in range(NP // CH):
        ohc = oh[c * CH:(c + 1) * CH]
        ranks.append(_dot(tri, ohc, (((1,), (0,)), ((), ()))) + carry)
        carry = carry + jnp.sum(ohc, axis=0, keepdims=True)
    rank = jnp.concatenate(ranks, axis=0)
    nbk = jnp.floor((carry + (BTM - 1)) * (1.0 / BTM))
    er_iota = jax.lax.broadcasted_iota(jnp.int32, (E, E), 0)
    ec_iota = jax.lax.broadcasted_iota(jnp.int32, (E, E), 1)
    tri_e = (er_iota < ec_iota).astype(jnp.float32)
    off = BTM * _dot(nbk, tri_e, (((1,), (0,)), ((), ())))
    pos_f = (jnp.sum(rank * oh, axis=1, keepdims=True)
             + jnp.sum(off * oh, axis=1, keepdims=True))
    pos_ref[...] = pos_f.astype(jnp.int32)
    nbk_ref[...] = nbk.astype(jnp.int32)


def _sc_dispatch_body(pos_hbm, xf_hbm, xs_hbm, idx_v, rows_v, sem, sem2):
    wid = lax.axis_index("s") * _SC_NC + lax.axis_index("c")
    pairs_per_w = NP // _SC_NW
    tok_shift = jnp.where(wid < _SC_NW // 2, 0, S)
    for s_ in range(pairs_per_w // 64):
        j0 = wid * pairs_per_w + s_ * 64
        cp1 = pltpu.async_copy(xf_hbm.at[pl.ds(j0 - tok_shift, 64)], rows_v,
                               sem)
        cp2 = pltpu.async_copy(pos_hbm.at[pl.ds(j0, 64)], idx_v, sem2)
        cp1.wait()
        cp2.wait()
        pltpu.async_copy(rows_v, xs_hbm.at[idx_v], sem).wait()


def _sc_combine_body(pos_hbm, of_hbm, a0_hbm, a1_hbm, idx_v, rows_v, sem,
                     sem2):
    wid = lax.axis_index("s") * _SC_NC + lax.axis_index("c")
    tok_per_w = S // _SC_NW
    off = wid * tok_per_w
    for part in range(2):
        out_hbm = a0_hbm if part == 0 else a1_hbm
        pltpu.async_copy(pos_hbm.at[pl.ds(part * S + off, tok_per_w)], idx_v,
                         sem2).wait()
        pltpu.async_copy(of_hbm.at[idx_v], rows_v, sem).wait()
        pltpu.async_copy(rows_v, out_hbm.at[pl.ds(off, tok_per_w)], sem).wait()


def _ffn_body(eid_ref, x_ref, wg_ref, wu_ref, wd_ref, o_ref):
    x = x_ref[...].astype(jnp.bfloat16)
    g = _dot(x, wg_ref[0], (((1,), (1,)), ((), ())))
    u = _dot(x, wu_ref[0], (((1,), (1,)), ((), ())))
    hexp = ((g * jax.nn.sigmoid(g)) * u).astype(jnp.bfloat16)
    o_ref[...] = _dot(hexp, wd_ref[0], (((1,), (1,)), ((), ())))


def _combine_body(res_ref, a0_ref, a1_ref, w0_ref, w1_ref, o_ref):
    o_ref[...] = (res_ref[...]
                  + w0_ref[...] * a0_ref[...]
                  + w1_ref[...] * a1_ref[...])


def kernel(hidden_states, start_pos, position_cos, position_sin, attention_mask,
           Wq, Wk, Wv, Wo, q_norm_w, k_norm_w, input_ln_w, post_ln_w,
           gate_w, Wg, Wu, Wd):
    x2d = hidden_states.reshape(S, D)
    wqkv = jnp.concatenate([Wq, Wk, Wv], axis=0).astype(jnp.bfloat16)
    wo3 = Wo.reshape(D, H, HD).astype(jnp.bfloat16)
    gate_b = gate_w.astype(jnp.bfloat16)
    wg_b = Wg.astype(jnp.bfloat16)
    wu_b = Wu.astype(jnp.bfloat16)
    wd_b = Wd.astype(jnp.bfloat16)
    lnw = input_ln_w.reshape(1, D)
    plw = post_ln_w.reshape(1, D)
    qnw = q_norm_w.reshape(1, HD)
    knw = k_norm_w.reshape(1, HD)

    q3, k3, v3 = pl.pallas_call(
        _qkv_body,
        grid=(NB,),
        in_specs=[
            pl.BlockSpec((BT, D), lambda i: (i, 0)),
            pl.BlockSpec((1, D), lambda i: (0, 0)),
            pl.BlockSpec(((H + 2 * KVH) * HD, D), lambda i: (0, 0)),
            pl.BlockSpec((BT, HD), lambda i: (i, 0)),
            pl.BlockSpec((BT, HD), lambda i: (i, 0)),
            pl.BlockSpec((1, HD), lambda i: (0, 0)),
            pl.BlockSpec((1, HD), lambda i: (0, 0)),
        ],
        out_specs=[
            pl.BlockSpec((H, BT, HD), lambda i: (0, i, 0)),
            pl.BlockSpec((KVH, BT, HD), lambda i: (0, i, 0)),
            pl.BlockSpec((KVH, BT, HD), lambda i: (0, i, 0)),
        ],
        out_shape=[
            jax.ShapeDtypeStruct((H, S, HD), jnp.bfloat16),
            jax.ShapeDtypeStruct((KVH, S, HD), jnp.bfloat16),
            jax.ShapeDtypeStruct((KVH, S, HD), jnp.bfloat16),
        ],
    )(x2d, lnw, wqkv, position_cos, position_sin, qnw, knw)

    attn3 = pl.pallas_call(
        _attn_body,
        grid=(H, NB),
        in_specs=[
            pl.BlockSpec((1, BT, HD), lambda h, i: (h, i, 0)),
            pl.BlockSpec((1, S, HD), lambda h, i: (h // (H // KVH), 0, 0)),
            pl.BlockSpec((1, S, HD), lambda h, i: (h // (H // KVH), 0, 0)),
        ],
        out_specs=pl.BlockSpec((1, BT, HD), lambda h, i: (h, i, 0)),
        out_shape=jax.ShapeDtypeStruct((H, S, HD), jnp.bfloat16),
    )(q3, k3, v3)

    hres, xf, logits = pl.pallas_call(
        _post_body,
        grid=(NB,),
        in_specs=[
            pl.BlockSpec((H, BT, HD), lambda i: (0, i, 0)),
            pl.BlockSpec((D, H, HD), lambda i: (0, 0, 0)),
            pl.BlockSpec((BT, D), lambda i: (i, 0)),
            pl.BlockSpec((1, D), lambda i: (0, 0)),
            pl.BlockSpec((E, D), lambda i: (0, 0)),
        ],
        out_specs=[
            pl.BlockSpec((BT, D), lambda i: (i, 0)),
            pl.BlockSpec((BT, D), lambda i: (i, 0)),
            pl.BlockSpec((BT, E), lambda i: (i, 0)),
        ],
        out_shape=[
            jax.ShapeDtypeStruct((S, D), jnp.float32),
            jax.ShapeDtypeStruct((S, D), jnp.float32),
            jax.ShapeDtypeStruct((S, E), jnp.float32),
        ],
    )(attn3, wo3, x2d, plw, gate_b)

    w0, w1, pos, nbk = pl.pallas_call(
        _route_body,
        grid=(1,),
        in_specs=[pl.BlockSpec((S, E), lambda i: (0, 0))],
        out_specs=[
            pl.BlockSpec((S, 1), lambda i: (0, 0)),
            pl.BlockSpec((S, 1), lambda i: (0, 0)),
            pl.BlockSpec((NP, 1), lambda i: (0, 0)),
            pl.BlockSpec((1, E), lambda i: (0, 0)),
        ],
        out_shape=[
            jax.ShapeDtypeStruct((S, 1), jnp.float32),
            jax.ShapeDtypeStruct((S, 1), jnp.float32),
            jax.ShapeDtypeStruct((NP, 1), jnp.int32),
            jax.ShapeDtypeStruct((1, E), jnp.int32),
        ],
    )(logits)

    pos1d = pos.reshape(NP)
    cnb = jnp.cumsum(nbk[0])
    eid = jnp.minimum(
        jnp.sum(cnb[:, None] <= jnp.arange(NG)[None, :], axis=0),
        E - 1).astype(jnp.int32)

    mesh = plsc.VectorSubcoreMesh(core_axis_name="c", subcore_axis_name="s")

    xs = pl.kernel(
        _sc_dispatch_body,
        mesh=mesh,
        out_type=jax.ShapeDtypeStruct((P, D), jnp.float32),
        scratch_types=[
            pltpu.VMEM((64,), jnp.int32),
            pltpu.VMEM((64, D), jnp.float32),
            pltpu.SemaphoreType.DMA,
            pltpu.SemaphoreType.DMA,
        ],
    )(pos1d, xf)

    osorted = pl.pallas_call(
        _ffn_body,
        grid_spec=pltpu.PrefetchScalarGridSpec(
            num_scalar_prefetch=1,
            grid=(NG,),
            in_specs=[
                pl.BlockSpec((BTM, D), lambda i, eid_ref: (i, 0)),
                pl.BlockSpec((1, F, D), lambda i, eid_ref: (eid_ref[i], 0, 0)),
                pl.BlockSpec((1, F, D), lambda i, eid_ref: (eid_ref[i], 0, 0)),
                pl.BlockSpec((1, D, F), lambda i, eid_ref: (eid_ref[i], 0, 0)),
            ],
            out_specs=pl.BlockSpec((BTM, D), lambda i, eid_ref: (i, 0)),
        ),
        out_shape=jax.ShapeDtypeStruct((P, D), jnp.float32),
    )(eid, xs, wg_b, wu_b, wd_b)

    a0, a1 = pl.kernel(
        _sc_combine_body,
        mesh=mesh,
        out_type=[
            jax.ShapeDtypeStruct((S, D), jnp.float32),
            jax.ShapeDtypeStruct((S, D), jnp.float32),
        ],
        scratch_types=[
            pltpu.VMEM((S // _SC_NW,), jnp.int32),
            pltpu.VMEM((S // _SC_NW, D), jnp.float32),
            pltpu.SemaphoreType.DMA,
            pltpu.SemaphoreType.DMA,
        ],
    )(pos1d, osorted)

    out = pl.pallas_call(
        _combine_body,
        grid=(NB,),
        in_specs=[
            pl.BlockSpec((BT, D), lambda i: (i, 0)),
            pl.BlockSpec((BT, D), lambda i: (i, 0)),
            pl.BlockSpec((BT, D), lambda i: (i, 0)),
            pl.BlockSpec((BT, 1), lambda i: (i, 0)),
            pl.BlockSpec((BT, 1), lambda i: (i, 0)),
        ],
        out_specs=pl.BlockSpec((BT, D), lambda i: (i, 0)),
        out_shape=jax.ShapeDtypeStruct((S, D), jnp.float32),
    )(hres, a0, a1, w0, w1)

    return out.reshape(B, S, D)


# speedup vs baseline: 1.9946x; 1.1479x over previous
import functools

import jax
import jax.numpy as jnp
from jax import lax
from jax.experimental import pallas as pl
from jax.experimental.pallas import tpu as pltpu
from jax.experimental.pallas import tpu_sc as plsc

B, S, D = 1, 2048, 1024
H, KVH, HD = 16, 4, 64
E, TOPK, F = 8, 2, 768
EPS = 1e-06

BT = 256
NB = S // BT

NP = TOPK * S
BTM = 128
P = NP + E * BTM
NG = P // BTM

_dot = functools.partial(jax.lax.dot_general, preferred_element_type=jnp.float32)

_SC_NC, _SC_NS = 2, 16
_SC_NW = _SC_NC * _SC_NS


def _qkv_body(h_ref, lnw_ref, wqkv_ref, cosq_ref, sinq_ref, cosk_ref,
              sink_ref, qnw_ref, knw_ref, mgq_ref, rq_ref, mgk_ref, rk_ref,
              q_ref, k_ref, v_ref):
    h32 = h_ref[...]
    var = jnp.mean(h32 * h32, axis=1, keepdims=True)
    hn = (h32 * jax.lax.rsqrt(var + EPS)) * lnw_ref[...]
    qkv = _dot(hn.astype(jnp.bfloat16), wqkv_ref[...], (((1,), (1,)), ((), ())))

    def headnorm_rope(x, mg_ref, r_ref, w_ref, cos, sin, scale):
        xb = x.astype(jnp.bfloat16)
        msum = _dot(xb * xb, mg_ref[...], (((1,), (0,)), ((), ())))
        xn = (x * jax.lax.rsqrt(msum * (1.0 / HD) + EPS)) * w_ref[...]
        xr = _dot(xn.astype(jnp.bfloat16), r_ref[...], (((1,), (0,)), ((), ())))
        return ((xn * cos + xr * sin) * scale).astype(jnp.bfloat16)

    q = qkv[:, :H * HD]
    k = qkv[:, H * HD:(H + KVH) * HD]
    v = qkv[:, (H + KVH) * HD:]
    q_ref[...] = headnorm_rope(q, mgq_ref, rq_ref, qnw_ref, cosq_ref[...],
                               sinq_ref[...], HD ** -0.5)
    k_ref[...] = headnorm_rope(k, mgk_ref, rk_ref, knw_ref, cosk_ref[...],
                               sink_ref[...], 1.0)
    v_ref[...] = v.astype(jnp.bfloat16)


def _attn_body(q_ref, k_ref, v_ref, o_ref):
    hp = pl.program_id(0)
    kv_sel = (hp // 2) % 2
    kp = k_ref[...]
    vp = v_ref[...]
    kk = jnp.where(kv_sel == 0, kp[:, :HD], kp[:, HD:])
    vv = jnp.where(kv_sel == 0, vp[:, :HD], vp[:, HD:])
    outs = []
    for sub in range(2):
        qh = q_ref[:, sub * HD:(sub + 1) * HD]
        s = _dot(qh, kk, (((1,), (1,)), ((), ())))
        m = jnp.max(s, axis=1, keepdims=True)
        p = jnp.exp(s - m)
        l = jnp.sum(p, axis=1, keepdims=True)
        o = _dot(p.astype(jnp.bfloat16), vv, (((1,), (0,)), ((), ())))
        outs.append(o / l)
    o_ref[...] = jnp.concatenate(outs, axis=1).astype(jnp.bfloat16)


def _post_body(attn_ref, wo_ref, res_ref, plw_ref, gate_ref,
               h_ref, x_ref, lg_ref):
    acc = _dot(attn_ref[...], wo_ref[...], (((1,), (1,)), ((), ())))
    hout = res_ref[...] + acc
    var = jnp.mean(hout * hout, axis=1, keepdims=True)
    xn = (hout * jax.lax.rsqrt(var + EPS)) * plw_ref[...]
    h_ref[...] = hout
    x_ref[...] = xn
    lg_ref[...] = _dot(xn, gate_ref[...], (((1,), (1,)), ((), ())))


def _route_body(lg_ref, w0_ref, w1_ref, pos_ref, nbk_ref):
    lg = lg_ref[...]
    m = jnp.max(lg, axis=1, keepdims=True)
    p = jnp.exp(lg - m)
    rw = p / jnp.sum(p, axis=1, keepdims=True)
    lanes = jax.lax.broadcasted_iota(jnp.int32, (S, E), 1)
    m0 = jnp.max(rw, axis=1, keepdims=True)
    i0 = jnp.min(jnp.where(rw == m0, lanes, E), axis=1, keepdims=True)
    sel0 = lanes == i0
    rw2 = jnp.where(sel0, -1.0, rw)
    m1 = jnp.max(rw2, axis=1, keepdims=True)
    i1 = jnp.min(jnp.where(rw2 == m1, lanes, E), axis=1, keepdims=True)
    sel1 = lanes == i1
    wsum = m0 + m1
    w0_ref[...] = m0 / wsum
    w1_ref[...] = m1 / wsum

    oh = jnp.concatenate([sel0.astype(jnp.float32), sel1.astype(jnp.float32)],
                         axis=0)
    CH = 512
    r_iota = jax.lax.broadcasted_iota(jnp.int32, (CH, CH), 0)
    c_iota = jax.lax.broadcasted_iota(jnp.int32, (CH, CH), 1)
    tri = (c_iota < r_iota).astype(jnp.float32)
    carry = jnp.zeros((1, E), jnp.float32)
    ranks = []
    for c in range(NP // CH):
        ohc = oh[c * CH:(c + 1) * CH]
        ranks.append(_dot(tri, ohc, (((1,), (0,)), ((), ()))) + carry)
        carry = carry + jnp.sum(ohc, axis=0, keepdims=True)
    rank = jnp.concatenate(ranks, axis=0)
    nbk = jnp.floor((carry + (BTM - 1)) * (1.0 / BTM))
    er_iota = jax.lax.broadcasted_iota(jnp.int32, (E, E), 0)
    ec_iota = jax.lax.broadcasted_iota(jnp.int32, (E, E), 1)
    tri_e = (er_iota < ec_iota).astype(jnp.float32)
    off = BTM * _dot(nbk, tri_e, (((1,), (0,)), ((), ())))
    pos_f = (jnp.sum(rank * oh, axis=1, keepdims=True)
             + jnp.sum(off * oh, axis=1, keepdims=True))
    pos_ref[...] = pos_f.astype(jnp.int32)
    nbk_ref[...] = nbk.astype(jnp.int32)


def _sc_dispatch_body(pos_hbm, xf_hbm, xs_hbm, idx_v, rows_v, sem, sem2):
    wid = lax.axis_index("s") * _SC_NC + lax.axis_index("c")
    pairs_per_w = NP // _SC_NW
    tok_shift = jnp.where(wid < _SC_NW // 2, 0, S)
    for s_ in range(pairs_per_w // 64):
        j0 = wid * pairs_per_w + s_ * 64
        cp1 = pltpu.async_copy(xf_hbm.at[pl.ds(j0 - tok_shift, 64)], rows_v,
                               sem)
        cp2 = pltpu.async_copy(pos_hbm.at[pl.ds(j0, 64)], idx_v, sem2)
        cp1.wait()
        cp2.wait()
        pltpu.async_copy(rows_v, xs_hbm.at[idx_v], sem).wait()


def _sc_combine_body(pos_hbm, of_hbm, a0_hbm, a1_hbm, idx_v, rows_v, sem,
                     sem2):
    wid = lax.axis_index("s") * _SC_NC + lax.axis_index("c")
    tok_per_w = S // _SC_NW
    off = wid * tok_per_w
    for part in range(2):
        out_hbm = a0_hbm if part == 0 else a1_hbm
        pltpu.async_copy(pos_hbm.at[pl.ds(part * S + off, tok_per_w)], idx_v,
                         sem2).wait()
        pltpu.async_copy(of_hbm.at[idx_v], rows_v, sem).wait()
        pltpu.async_copy(rows_v, out_hbm.at[pl.ds(off, tok_per_w)], sem).wait()


def _ffn_body(eid_ref, x_ref, wg_ref, wu_ref, wd_ref, o_ref):
    x = x_ref[...].astype(jnp.bfloat16)
    g = _dot(x, wg_ref[0], (((1,), (1,)), ((), ())))
    u = _dot(x, wu_ref[0], (((1,), (1,)), ((), ())))
    hexp = ((g * jax.nn.sigmoid(g)) * u).astype(jnp.bfloat16)
    o_ref[...] = _dot(hexp, wd_ref[0], (((1,), (1,)), ((), ())))


def _combine_body(res_ref, a0_ref, a1_ref, w0_ref, w1_ref, o_ref):
    o_ref[...] = (res_ref[...]
                  + w0_ref[...] * a0_ref[...]
                  + w1_ref[...] * a1_ref[...])


def kernel(hidden_states, start_pos, position_cos, position_sin, attention_mask,
           Wq, Wk, Wv, Wo, q_norm_w, k_norm_w, input_ln_w, post_ln_w,
           gate_w, Wg, Wu, Wd):
    x2d = hidden_states.reshape(S, D)
    wqkv = jnp.concatenate([Wq, Wk, Wv], axis=0).astype(jnp.bfloat16)
    wo2 = Wo.astype(jnp.bfloat16)
    wg_b = Wg.astype(jnp.bfloat16)
    wu_b = Wu.astype(jnp.bfloat16)
    wd_b = Wd.astype(jnp.bfloat16)
    lnw = input_ln_w.reshape(1, D)
    plw = post_ln_w.reshape(1, D)

    def rope_consts(nh):
        w = nh * HD
        jj = jnp.arange(w)[:, None]
        ll = jnp.arange(w)[None, :]
        g, p = ll // HD, ll % HD
        mg = (jj // HD == g).astype(jnp.bfloat16)
        r = (jnp.where((p < HD // 2) & (jj == g * HD + p + HD // 2), -1.0, 0.0)
             + jnp.where((p >= HD // 2) & (jj == g * HD + p - HD // 2),
                         1.0, 0.0)).astype(jnp.bfloat16)
        return mg, r

    mgq, rq = rope_consts(H)
    mgk, rk = rope_consts(KVH)
    cosq = jnp.tile(position_cos, (1, H))
    sinq = jnp.tile(position_sin, (1, H))
    cosk = jnp.tile(position_cos, (1, KVH))
    sink = jnp.tile(position_sin, (1, KVH))
    qnw = jnp.tile(q_norm_w, H).reshape(1, H * HD)
    knw = jnp.tile(k_norm_w, KVH).reshape(1, KVH * HD)

    QW, KW = H * HD, KVH * HD
    q2d, k2d, v2d = pl.pallas_call(
        _qkv_body,
        grid=(NB,),
        in_specs=[
            pl.BlockSpec((BT, D), lambda i: (i, 0)),
            pl.BlockSpec((1, D), lambda i: (0, 0)),
            pl.BlockSpec(((H + 2 * KVH) * HD, D), lambda i: (0, 0)),
            pl.BlockSpec((BT, QW), lambda i: (i, 0)),
            pl.BlockSpec((BT, QW), lambda i: (i, 0)),
            pl.BlockSpec((BT, KW), lambda i: (i, 0)),
            pl.BlockSpec((BT, KW), lambda i: (i, 0)),
            pl.BlockSpec((1, QW), lambda i: (0, 0)),
            pl.BlockSpec((1, KW), lambda i: (0, 0)),
            pl.BlockSpec((QW, QW), lambda i: (0, 0)),
            pl.BlockSpec((QW, QW), lambda i: (0, 0)),
            pl.BlockSpec((KW, KW), lambda i: (0, 0)),
            pl.BlockSpec((KW, KW), lambda i: (0, 0)),
        ],
        out_specs=[
            pl.BlockSpec((BT, QW), lambda i: (i, 0)),
            pl.BlockSpec((BT, KW), lambda i: (i, 0)),
            pl.BlockSpec((BT, KW), lambda i: (i, 0)),
        ],
        out_shape=[
            jax.ShapeDtypeStruct((S, QW), jnp.bfloat16),
            jax.ShapeDtypeStruct((S, KW), jnp.bfloat16),
            jax.ShapeDtypeStruct((S, KW), jnp.bfloat16),
        ],
        compiler_params=pltpu.CompilerParams(
            dimension_semantics=("parallel",)),
    )(x2d, lnw, wqkv, cosq, sinq, cosk, sink, qnw, knw, mgq, rq, mgk, rk)

    attn2d = pl.pallas_call(
        _attn_body,
        grid=(H // 2, NB),
        in_specs=[
            pl.BlockSpec((BT, 2 * HD), lambda hp, i: (i, hp)),
            pl.BlockSpec((S, 2 * HD), lambda hp, i: (0, hp // 4)),
            pl.BlockSpec((S, 2 * HD), lambda hp, i: (0, hp // 4)),
        ],
        out_specs=pl.BlockSpec((BT, 2 * HD), lambda hp, i: (i, hp)),
        out_shape=jax.ShapeDtypeStruct((S, QW), jnp.bfloat16),
        compiler_params=pltpu.CompilerParams(
            dimension_semantics=("parallel", "parallel")),
    )(q2d, k2d, v2d)

    hres, xf, logits = pl.pallas_call(
        _post_body,
        grid=(NB,),
        in_specs=[
            pl.BlockSpec((BT, QW), lambda i: (i, 0)),
            pl.BlockSpec((D, QW), lambda i: (0, 0)),
            pl.BlockSpec((BT, D), lambda i: (i, 0)),
            pl.BlockSpec((1, D), lambda i: (0, 0)),
            pl.BlockSpec((E, D), lambda i: (0, 0)),
        ],
        out_specs=[
            pl.BlockSpec((BT, D), lambda i: (i, 0)),
            pl.BlockSpec((BT, D), lambda i: (i, 0)),
            pl.BlockSpec((BT, E), lambda i: (i, 0)),
        ],
        out_shape=[
            jax.ShapeDtypeStruct((S, D), jnp.float32),
            jax.ShapeDtypeStruct((S, D), jnp.float32),
            jax.ShapeDtypeStruct((S, E), jnp.float32),
        ],
        compiler_params=pltpu.CompilerParams(
            dimension_semantics=("parallel",)),
    )(attn2d, wo2, x2d, plw, gate_w)

    w0, w1, pos, nbk = pl.pallas_call(
        _route_body,
        grid=(1,),
        in_specs=[pl.BlockSpec((S, E), lambda i: (0, 0))],
        out_specs=[
            pl.BlockSpec((S, 1), lambda i: (0, 0)),
            pl.BlockSpec((S, 1), lambda i: (0, 0)),
            pl.BlockSpec((NP, 1), lambda i: (0, 0)),
            pl.BlockSpec((1, E), lambda i: (0, 0)),
        ],
        out_shape=[
            jax.ShapeDtypeStruct((S, 1), jnp.float32),
            jax.ShapeDtypeStruct((S, 1), jnp.float32),
            jax.ShapeDtypeStruct((NP, 1), jnp.int32),
            jax.ShapeDtypeStruct((1, E), jnp.int32),
        ],
    )(logits)

    pos1d = pos.reshape(NP)
    cnb = jnp.cumsum(nbk[0])
    eid = jnp.minimum(
        jnp.sum(cnb[:, None] <= jnp.arange(NG)[None, :], axis=0),
        E - 1).astype(jnp.int32)

    mesh = plsc.VectorSubcoreMesh(core_axis_name="c", subcore_axis_name="s")

    xs = pl.kernel(
        _sc_dispatch_body,
        mesh=mesh,
        out_type=jax.ShapeDtypeStruct((P, D), jnp.float32),
        scratch_types=[
            pltpu.VMEM((64,), jnp.int32),
            pltpu.VMEM((64, D), jnp.float32),
            pltpu.SemaphoreType.DMA,
            pltpu.SemaphoreType.DMA,
        ],
    )(pos1d, xf)

    osorted = pl.pallas_call(
        _ffn_body,
        grid_spec=pltpu.PrefetchScalarGridSpec(
            num_scalar_prefetch=1,
            grid=(NG,),
            in_specs=[
                pl.BlockSpec((BTM, D), lambda i, eid_ref: (i, 0)),
                pl.BlockSpec((1, F, D), lambda i, eid_ref: (eid_ref[i], 0, 0)),
                pl.BlockSpec((1, F, D), lambda i, eid_ref: (eid_ref[i], 0, 0)),
                pl.BlockSpec((1, D, F), lambda i, eid_ref: (eid_ref[i], 0, 0)),
            ],
            out_specs=pl.BlockSpec((BTM, D), lambda i, eid_ref: (i, 0)),
        ),
        out_shape=jax.ShapeDtypeStruct((P, D), jnp.float32),
        compiler_params=pltpu.CompilerParams(
            dimension_semantics=("parallel",)),
    )(eid, xs, wg_b, wu_b, wd_b)

    a0, a1 = pl.kernel(
        _sc_combine_body,
        mesh=mesh,
        out_type=[
            jax.ShapeDtypeStruct((S, D), jnp.float32),
            jax.ShapeDtypeStruct((S, D), jnp.float32),
        ],
        scratch_types=[
            pltpu.VMEM((S // _SC_NW,), jnp.int32),
            pltpu.VMEM((S // _SC_NW, D), jnp.float32),
            pltpu.SemaphoreType.DMA,
            pltpu.SemaphoreType.DMA,
        ],
    )(pos1d, osorted)

    out = pl.pallas_call(
        _combine_body,
        grid=(NB,),
        in_specs=[
            pl.BlockSpec((BT, D), lambda i: (i, 0)),
            pl.BlockSpec((BT, D), lambda i: (i, 0)),
            pl.BlockSpec((BT, D), lambda i: (i, 0)),
            pl.BlockSpec((BT, 1), lambda i: (i, 0)),
            pl.BlockSpec((BT, 1), lambda i: (i, 0)),
        ],
        out_specs=pl.BlockSpec((BT, D), lambda i: (i, 0)),
        out_shape=jax.ShapeDtypeStruct((S, D), jnp.float32),
        compiler_params=pltpu.CompilerParams(
            dimension_semantics=("parallel",)),
    )(hres, a0, a1, w0, w1)

    return out.reshape(B, S, D)


# speedup vs baseline: 2.3150x; 1.1607x over previous
import functools

import jax
import jax.numpy as jnp
from jax import lax
from jax.experimental import pallas as pl
from jax.experimental.pallas import tpu as pltpu
from jax.experimental.pallas import tpu_sc as plsc

B, S, D = 1, 2048, 1024
H, KVH, HD = 16, 4, 64
E, TOPK, F = 8, 2, 768
EPS = 1e-06

BT = 512
NB = S // BT
BQ = 512

NP = TOPK * S
BTM = 128
P = NP + E * BTM
NG = P // BTM

_dot = functools.partial(jax.lax.dot_general, preferred_element_type=jnp.float32)

_SC_NC, _SC_NS = 2, 16
_SC_NW = _SC_NC * _SC_NS


def _qkv_body(h_ref, lnw_ref, wqkv_ref, cosq_ref, sinq_ref, cosk_ref,
              sink_ref, qnw_ref, knw_ref, mgq_ref, rq_ref, mgk_ref, rk_ref,
              q_ref, k_ref, v_ref):
    h32 = h_ref[...]
    var = jnp.mean(h32 * h32, axis=1, keepdims=True)
    hn = (h32 * jax.lax.rsqrt(var + EPS)) * lnw_ref[...]
    qkv = _dot(hn.astype(jnp.bfloat16), wqkv_ref[...], (((1,), (1,)), ((), ())))

    def headnorm_rope(x, mg_ref, r_ref, w_ref, cos_ref, sin_ref):
        xb = x.astype(jnp.bfloat16)
        msum = _dot(xb * xb, mg_ref[...], (((1,), (0,)), ((), ())))
        xn = (x * jax.lax.rsqrt(msum * (1.0 / HD) + EPS)) * w_ref[...]
        xr = _dot(xn.astype(jnp.bfloat16), r_ref[...], (((1,), (0,)), ((), ())))
        cos = cos_ref[...].astype(jnp.float32)
        sin = sin_ref[...].astype(jnp.float32)
        return (xn * cos + xr * sin).astype(jnp.bfloat16)

    q = qkv[:, :H * HD]
    k = qkv[:, H * HD:(H + KVH) * HD]
    v = qkv[:, (H + KVH) * HD:]
    q_ref[...] = headnorm_rope(q, mgq_ref, rq_ref, qnw_ref, cosq_ref, sinq_ref)
    k_ref[...] = headnorm_rope(k, mgk_ref, rk_ref, knw_ref, cosk_ref, sink_ref)
    v_ref[...] = v.astype(jnp.bfloat16)


def _attn_body(q_ref, k_ref, v_ref, o_ref):
    g = pl.program_id(0)
    kp = k_ref[...]
    vp = v_ref[...]
    kk = kp[:, :HD]
    vv = vp[:, :HD]
    for j in range(1, KVH):
        kk = jnp.where(g == j, kp[:, j * HD:(j + 1) * HD], kk)
        vv = jnp.where(g == j, vp[:, j * HD:(j + 1) * HD], vv)
    outs = []
    for sub in range(H // KVH):
        qh = q_ref[:, sub * HD:(sub + 1) * HD]
        s = _dot(qh, kk, (((1,), (1,)), ((), ())))
        m = jnp.max(s, axis=1, keepdims=True)
        p = jnp.exp(s - m)
        l = jnp.sum(p, axis=1, keepdims=True)
        o = _dot(p.astype(jnp.bfloat16), vv, (((1,), (0,)), ((), ())))
        outs.append(o / l)
    o_ref[...] = jnp.concatenate(outs, axis=1).astype(jnp.bfloat16)


def _post_body(attn_ref, wo_ref, res_ref, plw_ref, gate_ref,
               h_ref, x_ref, lg_ref):
    acc = _dot(attn_ref[...], wo_ref[...], (((1,), (1,)), ((), ())))
    hout = res_ref[...] + acc
    var = jnp.mean(hout * hout, axis=1, keepdims=True)
    xn = (hout * jax.lax.rsqrt(var + EPS)) * plw_ref[...]
    h_ref[...] = hout
    x_ref[...] = xn
    lg_ref[...] = _dot(xn, gate_ref[...], (((1,), (1,)), ((), ())))


def _route_body(lg_ref, w0_ref, w1_ref, pos_ref, nbk_ref):
    lg = lg_ref[...]
    m = jnp.max(lg, axis=1, keepdims=True)
    p = jnp.exp(lg - m)
    rw = p / jnp.sum(p, axis=1, keepdims=True)
    lanes = jax.lax.broadcasted_iota(jnp.int32, (S, E), 1)
    m0 = jnp.max(rw, axis=1, keepdims=True)
    i0 = jnp.min(jnp.where(rw == m0, lanes, E), axis=1, keepdims=True)
    sel0 = lanes == i0
    rw2 = jnp.where(sel0, -1.0, rw)
    m1 = jnp.max(rw2, axis=1, keepdims=True)
    i1 = jnp.min(jnp.where(rw2 == m1, lanes, E), axis=1, keepdims=True)
    sel1 = lanes == i1
    wsum = m0 + m1
    w0_ref[...] = m0 / wsum
    w1_ref[...] = m1 / wsum

    oh = jnp.concatenate([sel0.astype(jnp.float32), sel1.astype(jnp.float32)],
                         axis=0)
    CH = 512
    r_iota = jax.lax.broadcasted_iota(jnp.int32, (CH, CH), 0)
    c_iota = jax.lax.broadcasted_iota(jnp.int32, (CH, CH), 1)
    tri = (c_iota < r_iota).astype(jnp.float32)
    carry = jnp.zeros((1, E), jnp.float32)
    ranks = []
    for c in range(NP // CH):
        ohc = oh[c * CH:(c + 1) * CH]
        ranks.append(_dot(tri, ohc, (((1,), (0,)), ((), ()))) + carry)
        carry = carry + jnp.sum(ohc, axis=0, keepdims=True)
    rank = jnp.concatenate(ranks, axis=0)
    nbk = jnp.floor((carry + (BTM - 1)) * (1.0 / BTM))
    er_iota = jax.lax.broadcasted_iota(jnp.int32, (E, E), 0)
    ec_iota = jax.lax.broadcasted_iota(jnp.int32, (E, E), 1)
    tri_e = (er_iota < ec_iota).astype(jnp.float32)
    off = BTM * _dot(nbk, tri_e, (((1,), (0,)), ((), ())))
    pos_f = (jnp.sum(rank * oh, axis=1, keepdims=True)
             + jnp.sum(off * oh, axis=1, keepdims=True))
    pos_ref[...] = pos_f.astype(jnp.int32)
    nbk_ref[...] = nbk.astype(jnp.int32)


def _sc_dispatch_body(pos_hbm, xf_hbm, xs_hbm, idx_v, rows_v, sem, sem2):
    wid = lax.axis_index("s") * _SC_NC + lax.axis_index("c")
    pairs_per_w = NP // _SC_NW
    tok_shift = jnp.where(wid < _SC_NW // 2, 0, S)
    for s_ in range(pairs_per_w // 64):
        j0 = wid * pairs_per_w + s_ * 64
        cp1 = pltpu.async_copy(xf_hbm.at[pl.ds(j0 - tok_shift, 64)], rows_v,
                               sem)
        cp2 = pltpu.async_copy(pos_hbm.at[pl.ds(j0, 64)], idx_v, sem2)
        cp1.wait()
        cp2.wait()
        pltpu.async_copy(rows_v, xs_hbm.at[idx_v], sem).wait()


def _sc_combine_body(pos_hbm, of_hbm, a0_hbm, a1_hbm, idx_v, rows_v, sem,
                     sem2):
    wid = lax.axis_index("s") * _SC_NC + lax.axis_index("c")
    tok_per_w = S // _SC_NW
    off = wid * tok_per_w
    for part in range(2):
        out_hbm = a0_hbm if part == 0 else a1_hbm
        pltpu.async_copy(pos_hbm.at[pl.ds(part * S + off, tok_per_w)], idx_v,
                         sem2).wait()
        pltpu.async_copy(of_hbm.at[idx_v], rows_v, sem).wait()
        pltpu.async_copy(rows_v, out_hbm.at[pl.ds(off, tok_per_w)], sem).wait()


def _ffn_body(eid_ref, x_ref, wg_ref, wu_ref, wd_ref, o_ref):
    x = x_ref[...].astype(jnp.bfloat16)
    g = _dot(x, wg_ref[0], (((1,), (1,)), ((), ())))
    u = _dot(x, wu_ref[0], (((1,), (1,)), ((), ())))
    hexp = ((g * jax.nn.sigmoid(g)) * u).astype(jnp.bfloat16)
    o_ref[...] = _dot(hexp, wd_ref[0], (((1,), (1,)), ((), ())))


def _combine_body(res_ref, a0_ref, a1_ref, w0_ref, w1_ref, o_ref):
    o_ref[...] = (res_ref[...]
                  + w0_ref[...] * a0_ref[...]
                  + w1_ref[...] * a1_ref[...])


def kernel(hidden_states, start_pos, position_cos, position_sin, attention_mask,
           Wq, Wk, Wv, Wo, q_norm_w, k_norm_w, input_ln_w, post_ln_w,
           gate_w, Wg, Wu, Wd):
    x2d = hidden_states.reshape(S, D)
    wqkv = jnp.concatenate([Wq, Wk, Wv], axis=0).astype(jnp.bfloat16)
    wo2 = Wo.astype(jnp.bfloat16)
    wg_b = Wg.astype(jnp.bfloat16)
    wu_b = Wu.astype(jnp.bfloat16)
    wd_b = Wd.astype(jnp.bfloat16)
    lnw = input_ln_w.reshape(1, D)
    plw = post_ln_w.reshape(1, D)

    def rope_consts(nh):
        w = nh * HD
        jj = jnp.arange(w)[:, None]
        ll = jnp.arange(w)[None, :]
        g, p = ll // HD, ll % HD
        mg = (jj // HD == g).astype(jnp.bfloat16)
        r = (jnp.where((p < HD // 2) & (jj == g * HD + p + HD // 2), -1.0, 0.0)
             + jnp.where((p >= HD // 2) & (jj == g * HD + p - HD // 2),
                         1.0, 0.0)).astype(jnp.bfloat16)
        return mg, r

    mgq, rq = rope_consts(H)
    mgk, rk = rope_consts(KVH)
    cosq = (jnp.tile(position_cos, (1, H)) * (HD ** -0.5)).astype(jnp.bfloat16)
    sinq = (jnp.tile(position_sin, (1, H)) * (HD ** -0.5)).astype(jnp.bfloat16)
    cosk = jnp.tile(position_cos, (1, KVH)).astype(jnp.bfloat16)
    sink = jnp.tile(position_sin, (1, KVH)).astype(jnp.bfloat16)
    qnw = jnp.tile(q_norm_w, H).reshape(1, H * HD)
    knw = jnp.tile(k_norm_w, KVH).reshape(1, KVH * HD)

    QW, KW = H * HD, KVH * HD
    q2d, k2d, v2d = pl.pallas_call(
        _qkv_body,
        grid=(NB,),
        in_specs=[
            pl.BlockSpec((BT, D), lambda i: (i, 0)),
            pl.BlockSpec((1, D), lambda i: (0, 0)),
            pl.BlockSpec(((H + 2 * KVH) * HD, D), lambda i: (0, 0)),
            pl.BlockSpec((BT, QW), lambda i: (i, 0)),
            pl.BlockSpec((BT, QW), lambda i: (i, 0)),
            pl.BlockSpec((BT, KW), lambda i: (i, 0)),
            pl.BlockSpec((BT, KW), lambda i: (i, 0)),
            pl.BlockSpec((1, QW), lambda i: (0, 0)),
            pl.BlockSpec((1, KW), lambda i: (0, 0)),
            pl.BlockSpec((QW, QW), lambda i: (0, 0)),
            pl.BlockSpec((QW, QW), lambda i: (0, 0)),
            pl.BlockSpec((KW, KW), lambda i: (0, 0)),
            pl.BlockSpec((KW, KW), lambda i: (0, 0)),
        ],
        out_specs=[
            pl.BlockSpec((BT, QW), lambda i: (i, 0)),
            pl.BlockSpec((BT, KW), lambda i: (i, 0)),
            pl.BlockSpec((BT, KW), lambda i: (i, 0)),
        ],
        out_shape=[
            jax.ShapeDtypeStruct((S, QW), jnp.bfloat16),
            jax.ShapeDtypeStruct((S, KW), jnp.bfloat16),
            jax.ShapeDtypeStruct((S, KW), jnp.bfloat16),
        ],
        compiler_params=pltpu.CompilerParams(
            dimension_semantics=("parallel",)),
    )(x2d, lnw, wqkv, cosq, sinq, cosk, sink, qnw, knw, mgq, rq, mgk, rk)

    GW = (H // KVH) * HD
    attn2d = pl.pallas_call(
        _attn_body,
        grid=(KVH, S // BQ),
        in_specs=[
            pl.BlockSpec((BQ, GW), lambda g, i: (i, g)),
            pl.BlockSpec((S, KW), lambda g, i: (0, 0)),
            pl.BlockSpec((S, KW), lambda g, i: (0, 0)),
        ],
        out_specs=pl.BlockSpec((BQ, GW), lambda g, i: (i, g)),
        out_shape=jax.ShapeDtypeStruct((S, QW), jnp.bfloat16),
        compiler_params=pltpu.CompilerParams(
            dimension_semantics=("parallel", "parallel")),
    )(q2d, k2d, v2d)

    hres, xf, logits = pl.pallas_call(
        _post_body,
        grid=(NB,),
        in_specs=[
            pl.BlockSpec((BT, QW), lambda i: (i, 0)),
            pl.BlockSpec((D, QW), lambda i: (0, 0)),
            pl.BlockSpec((BT, D), lambda i: (i, 0)),
            pl.BlockSpec((1, D), lambda i: (0, 0)),
            pl.BlockSpec((E, D), lambda i: (0, 0)),
        ],
        out_specs=[
            pl.BlockSpec((BT, D), lambda i: (i, 0)),
            pl.BlockSpec((BT, D), lambda i: (i, 0)),
            pl.BlockSpec((BT, E), lambda i: (i, 0)),
        ],
        out_shape=[
            jax.ShapeDtypeStruct((S, D), jnp.float32),
            jax.ShapeDtypeStruct((S, D), jnp.float32),
            jax.ShapeDtypeStruct((S, E), jnp.float32),
        ],
        compiler_params=pltpu.CompilerParams(
            dimension_semantics=("parallel",)),
    )(attn2d, wo2, x2d, plw, gate_w)

    w0, w1, pos, nbk = pl.pallas_call(
        _route_body,
        grid=(1,),
        in_specs=[pl.BlockSpec((S, E), lambda i: (0, 0))],
        out_specs=[
            pl.BlockSpec((S, 1), lambda i: (0, 0)),
            pl.BlockSpec((S, 1), lambda i: (0, 0)),
            pl.BlockSpec((NP, 1), lambda i: (0, 0)),
            pl.BlockSpec((1, E), lambda i: (0, 0)),
        ],
        out_shape=[
            jax.ShapeDtypeStruct((S, 1), jnp.float32),
            jax.ShapeDtypeStruct((S, 1), jnp.float32),
            jax.ShapeDtypeStruct((NP, 1), jnp.int32),
            jax.ShapeDtypeStruct((1, E), jnp.int32),
        ],
    )(logits)

    pos1d = pos.reshape(NP)
    cnb = jnp.cumsum(nbk[0])
    eid = jnp.minimum(
        jnp.sum(cnb[:, None] <= jnp.arange(NG)[None, :], axis=0),
        E - 1).astype(jnp.int32)

    mesh = plsc.VectorSubcoreMesh(core_axis_name="c", subcore_axis_name="s")

    xs = pl.kernel(
        _sc_dispatch_body,
        mesh=mesh,
        out_type=jax.ShapeDtypeStruct((P, D), jnp.float32),
        scratch_types=[
            pltpu.VMEM((64,), jnp.int32),
            pltpu.VMEM((64, D), jnp.float32),
            pltpu.SemaphoreType.DMA,
            pltpu.SemaphoreType.DMA,
        ],
    )(pos1d, xf)

    osorted = pl.pallas_call(
        _ffn_body,
        grid_spec=pltpu.PrefetchScalarGridSpec(
            num_scalar_prefetch=1,
            grid=(NG,),
            in_specs=[
                pl.BlockSpec((BTM, D), lambda i, eid_ref: (i, 0)),
                pl.BlockSpec((1, F, D), lambda i, eid_ref: (eid_ref[i], 0, 0)),
                pl.BlockSpec((1, F, D), lambda i, eid_ref: (eid_ref[i], 0, 0)),
                pl.BlockSpec((1, D, F), lambda i, eid_ref: (eid_ref[i], 0, 0)),
            ],
            out_specs=pl.BlockSpec((BTM, D), lambda i, eid_ref: (i, 0)),
        ),
        out_shape=jax.ShapeDtypeStruct((P, D), jnp.float32),
        compiler_params=pltpu.CompilerParams(
            dimension_semantics=("parallel",)),
    )(eid, xs, wg_b, wu_b, wd_b)

    a0, a1 = pl.kernel(
        _sc_combine_body,
        mesh=mesh,
        out_type=[
            jax.ShapeDtypeStruct((S, D), jnp.float32),
            jax.ShapeDtypeStruct((S, D), jnp.float32),
        ],
        scratch_types=[
            pltpu.VMEM((S // _SC_NW,), jnp.int32),
            pltpu.VMEM((S // _SC_NW, D), jnp.float32),
            pltpu.SemaphoreType.DMA,
            pltpu.SemaphoreType.DMA,
        ],
    )(pos1d, osorted)

    out = pl.pallas_call(
        _combine_body,
        grid=(NB,),
        in_specs=[
            pl.BlockSpec((BT, D), lambda i: (i, 0)),
            pl.BlockSpec((BT, D), lambda i: (i, 0)),
            pl.BlockSpec((BT, D), lambda i: (i, 0)),
            pl.BlockSpec((BT, 1), lambda i: (i, 0)),
            pl.BlockSpec((BT, 1), lambda i: (i, 0)),
        ],
        out_specs=pl.BlockSpec((BT, D), lambda i: (i, 0)),
        out_shape=jax.ShapeDtypeStruct((S, D), jnp.float32),
        compiler_params=pltpu.CompilerParams(
            dimension_semantics=("parallel",)),
    )(hres, a0, a1, w0, w1)

    return out.reshape(B, S, D)


# speedup vs baseline: 2.5697x; 1.1100x over previous
import functools

import jax
import jax.numpy as jnp
from jax import lax
from jax.experimental import pallas as pl
from jax.experimental.pallas import tpu as pltpu
from jax.experimental.pallas import tpu_sc as plsc

B, S, D = 1, 2048, 1024
H, KVH, HD = 16, 4, 64
E, TOPK, F = 8, 2, 768
EPS = 1e-06

BT = 512
NB = S // BT
BQ = 512

NP = TOPK * S
BTM = 128
P = NP + E * BTM
NG = P // BTM

_dot = functools.partial(jax.lax.dot_general, preferred_element_type=jnp.float32)

_SC_NC, _SC_NS = 2, 16
_SC_NW = _SC_NC * _SC_NS


def _qkv_body(h_ref, lnw_ref, wqkv_ref, cos_ref, sin_ref, qnw_ref, knw_ref,
              mgq_ref, rq_ref, tq_ref, mgk_ref, rk_ref, tk_ref,
              q_ref, k_ref, v_ref):
    h32 = h_ref[...]
    var = jnp.mean(h32 * h32, axis=1, keepdims=True)
    hn = (h32 * jax.lax.rsqrt(var + EPS)) * lnw_ref[...]
    qkv = _dot(hn.astype(jnp.bfloat16), wqkv_ref[...], (((1,), (1,)), ((), ())))
    cosb = cos_ref[...]
    sinb = sin_ref[...]

    def headnorm_rope(x, mg_ref, r_ref, w_ref, t_ref):
        xb = x.astype(jnp.bfloat16)
        msum = _dot(xb * xb, mg_ref[...], (((1,), (0,)), ((), ())))
        xn = (x * jax.lax.rsqrt(msum * (1.0 / HD) + EPS)) * w_ref[...]
        xr = _dot(xn.astype(jnp.bfloat16), r_ref[...], (((1,), (0,)), ((), ())))
        cos = _dot(cosb, t_ref[...], (((1,), (0,)), ((), ())))
        sin = _dot(sinb, t_ref[...], (((1,), (0,)), ((), ())))
        return (xn * cos + xr * sin).astype(jnp.bfloat16)

    q = qkv[:, :H * HD]
    k = qkv[:, H * HD:(H + KVH) * HD]
    v = qkv[:, (H + KVH) * HD:]
    q_ref[...] = headnorm_rope(q, mgq_ref, rq_ref, qnw_ref, tq_ref)
    k_ref[...] = headnorm_rope(k, mgk_ref, rk_ref, knw_ref, tk_ref)
    v_ref[...] = v.astype(jnp.bfloat16)


def _attn_body(q_ref, k_ref, v_ref, o_ref):
    g = pl.program_id(0)
    kp = k_ref[...]
    vp = v_ref[...]
    kk = kp[:, :HD]
    vv = vp[:, :HD]
    for j in range(1, KVH):
        kk = jnp.where(g == j, kp[:, j * HD:(j + 1) * HD], kk)
        vv = jnp.where(g == j, vp[:, j * HD:(j + 1) * HD], vv)
    outs = []
    for sub in range(H // KVH):
        qh = q_ref[:, sub * HD:(sub + 1) * HD]
        s = _dot(qh, kk, (((1,), (1,)), ((), ())))
        p = jnp.exp(s)
        l = jnp.sum(p, axis=1, keepdims=True)
        o = _dot(p.astype(jnp.bfloat16), vv, (((1,), (0,)), ((), ())))
        outs.append(o / l)
    o_ref[...] = jnp.concatenate(outs, axis=1).astype(jnp.bfloat16)


def _post_body(attn_ref, wo_ref, res_ref, plw_ref, gate_ref,
               h_ref, x_ref, lg_ref):
    acc = _dot(attn_ref[...], wo_ref[...], (((1,), (1,)), ((), ())))
    hout = res_ref[...] + acc
    var = jnp.mean(hout * hout, axis=1, keepdims=True)
    xn = (hout * jax.lax.rsqrt(var + EPS)) * plw_ref[...]
    h_ref[...] = hout
    x_ref[...] = xn
    lg_ref[...] = _dot(xn, gate_ref[...], (((1,), (1,)), ((), ())))


def _route_body(lg_ref, w0_ref, w1_ref, pos_ref, nbk_ref):
    lg = lg_ref[...]
    m = jnp.max(lg, axis=1, keepdims=True)
    p = jnp.exp(lg - m)
    rw = p / jnp.sum(p, axis=1, keepdims=True)
    lanes = jax.lax.broadcasted_iota(jnp.int32, (S, E), 1)
    m0 = jnp.max(rw, axis=1, keepdims=True)
    i0 = jnp.min(jnp.where(rw == m0, lanes, E), axis=1, keepdims=True)
    sel0 = lanes == i0
    rw2 = jnp.where(sel0, -1.0, rw)
    m1 = jnp.max(rw2, axis=1, keepdims=True)
    i1 = jnp.min(jnp.where(rw2 == m1, lanes, E), axis=1, keepdims=True)
    sel1 = lanes == i1
    wsum = m0 + m1
    w0_ref[...] = m0 / wsum
    w1_ref[...] = m1 / wsum

    oh = jnp.concatenate([sel0.astype(jnp.float32), sel1.astype(jnp.float32)],
                         axis=0)
    CH = 512
    r_iota = jax.lax.broadcasted_iota(jnp.int32, (CH, CH), 0)
    c_iota = jax.lax.broadcasted_iota(jnp.int32, (CH, CH), 1)
    tri = (c_iota < r_iota).astype(jnp.float32)
    carry = jnp.zeros((1, E), jnp.float32)
    ranks = []
    for c in range(NP // CH):
        ohc = oh[c * CH:(c + 1) * CH]
        ranks.append(_dot(tri, ohc, (((1,), (0,)), ((), ()))) + carry)
        carry = carry + jnp.sum(ohc, axis=0, keepdims=True)
    rank = jnp.concatenate(ranks, axis=0)
    nbk = jnp.floor((carry + (BTM - 1)) * (1.0 / BTM))
    er_iota = jax.lax.broadcasted_iota(jnp.int32, (E, E), 0)
    ec_iota = jax.lax.broadcasted_iota(jnp.int32, (E, E), 1)
    tri_e = (er_iota < ec_iota).astype(jnp.float32)
    off = BTM * _dot(nbk, tri_e, (((1,), (0,)), ((), ())))
    pos_f = (jnp.sum(rank * oh, axis=1, keepdims=True)
             + jnp.sum(off * oh, axis=1, keepdims=True))
    pos_ref[...] = pos_f.astype(jnp.int32)
    nbk_ref[...] = nbk.astype(jnp.int32)


def _sc_dispatch_body(pos_hbm, xf_hbm, xs_hbm, idx_v, rows_v, sem, sem2):
    wid = lax.axis_index("s") * _SC_NC + lax.axis_index("c")
    pairs_per_w = NP // _SC_NW
    tok_shift = jnp.where(wid < _SC_NW // 2, 0, S)
    for s_ in range(pairs_per_w // 64):
        j0 = wid * pairs_per_w + s_ * 64
        cp1 = pltpu.async_copy(xf_hbm.at[pl.ds(j0 - tok_shift, 64)], rows_v,
                               sem)
        cp2 = pltpu.async_copy(pos_hbm.at[pl.ds(j0, 64)], idx_v, sem2)
        cp1.wait()
        cp2.wait()
        pltpu.async_copy(rows_v, xs_hbm.at[idx_v], sem).wait()


def _sc_combine_body(pos_hbm, of_hbm, a0_hbm, a1_hbm, idx_v, rows_v, sem,
                     sem2):
    wid = lax.axis_index("s") * _SC_NC + lax.axis_index("c")
    tok_per_w = S // _SC_NW
    off = wid * tok_per_w
    for part in range(2):
        out_hbm = a0_hbm if part == 0 else a1_hbm
        pltpu.async_copy(pos_hbm.at[pl.ds(part * S + off, tok_per_w)], idx_v,
                         sem2).wait()
        pltpu.async_copy(of_hbm.at[idx_v], rows_v, sem).wait()
        pltpu.async_copy(rows_v, out_hbm.at[pl.ds(off, tok_per_w)], sem).wait()


def _ffn_body(eid_ref, x_ref, wg_ref, wu_ref, wd_ref, o_ref):
    x = x_ref[...].astype(jnp.bfloat16)
    g = _dot(x, wg_ref[0], (((1,), (1,)), ((), ())))
    u = _dot(x, wu_ref[0], (((1,), (1,)), ((), ())))
    hexp = ((g * jax.nn.sigmoid(g)) * u).astype(jnp.bfloat16)
    o_ref[...] = _dot(hexp, wd_ref[0], (((1,), (1,)), ((), ())))


def _combine_body(res_ref, a0_ref, a1_ref, w0_ref, w1_ref, o_ref):
    o_ref[...] = (res_ref[...]
                  + w0_ref[...] * a0_ref[...]
                  + w1_ref[...] * a1_ref[...])


def kernel(hidden_states, start_pos, position_cos, position_sin, attention_mask,
           Wq, Wk, Wv, Wo, q_norm_w, k_norm_w, input_ln_w, post_ln_w,
           gate_w, Wg, Wu, Wd):
    x2d = hidden_states.reshape(S, D)
    wqkv = jnp.concatenate([Wq, Wk, Wv], axis=0).astype(jnp.bfloat16)
    wo2 = Wo.astype(jnp.bfloat16)
    wg_b = Wg.astype(jnp.bfloat16)
    wu_b = Wu.astype(jnp.bfloat16)
    wd_b = Wd.astype(jnp.bfloat16)
    lnw = input_ln_w.reshape(1, D)
    plw = post_ln_w.reshape(1, D)

    def rope_consts(nh):
        w = nh * HD
        jj = jnp.arange(w)[:, None]
        ll = jnp.arange(w)[None, :]
        g, p = ll // HD, ll % HD
        mg = (jj // HD == g).astype(jnp.bfloat16)
        r = (jnp.where((p < HD // 2) & (jj == g * HD + p + HD // 2), -1.0, 0.0)
             + jnp.where((p >= HD // 2) & (jj == g * HD + p - HD // 2),
                         1.0, 0.0)).astype(jnp.bfloat16)
        return mg, r

    mgq, rq = rope_consts(H)
    mgk, rk = rope_consts(KVH)
    jhd = jnp.arange(HD)[:, None]
    tq = ((jnp.arange(H * HD)[None, :] % HD == jhd)
          * (HD ** -0.5)).astype(jnp.bfloat16)
    tk = (jnp.arange(KVH * HD)[None, :] % HD == jhd).astype(jnp.bfloat16)
    cosb = position_cos.astype(jnp.bfloat16)
    sinb = position_sin.astype(jnp.bfloat16)
    qnw = jnp.tile(q_norm_w, H).reshape(1, H * HD)
    knw = jnp.tile(k_norm_w, KVH).reshape(1, KVH * HD)

    QW, KW = H * HD, KVH * HD
    q2d, k2d, v2d = pl.pallas_call(
        _qkv_body,
        grid=(NB,),
        in_specs=[
            pl.BlockSpec((BT, D), lambda i: (i, 0)),
            pl.BlockSpec((1, D), lambda i: (0, 0)),
            pl.BlockSpec(((H + 2 * KVH) * HD, D), lambda i: (0, 0)),
            pl.BlockSpec((BT, HD), lambda i: (i, 0)),
            pl.BlockSpec((BT, HD), lambda i: (i, 0)),
            pl.BlockSpec((1, QW), lambda i: (0, 0)),
            pl.BlockSpec((1, KW), lambda i: (0, 0)),
            pl.BlockSpec((QW, QW), lambda i: (0, 0)),
            pl.BlockSpec((QW, QW), lambda i: (0, 0)),
            pl.BlockSpec((HD, QW), lambda i: (0, 0)),
            pl.BlockSpec((KW, KW), lambda i: (0, 0)),
            pl.BlockSpec((KW, KW), lambda i: (0, 0)),
            pl.BlockSpec((HD, KW), lambda i: (0, 0)),
        ],
        out_specs=[
            pl.BlockSpec((BT, QW), lambda i: (i, 0)),
            pl.BlockSpec((BT, KW), lambda i: (i, 0)),
            pl.BlockSpec((BT, KW), lambda i: (i, 0)),
        ],
        out_shape=[
            jax.ShapeDtypeStruct((S, QW), jnp.bfloat16),
            jax.ShapeDtypeStruct((S, KW), jnp.bfloat16),
            jax.ShapeDtypeStruct((S, KW), jnp.bfloat16),
        ],
        compiler_params=pltpu.CompilerParams(
            dimension_semantics=("parallel",)),
    )(x2d, lnw, wqkv, cosb, sinb, qnw, knw, mgq, rq, tq, mgk, rk, tk)

    GW = (H // KVH) * HD
    attn2d = pl.pallas_call(
        _attn_body,
        grid=(KVH, S // BQ),
        in_specs=[
            pl.BlockSpec((BQ, GW), lambda g, i: (i, g)),
            pl.BlockSpec((S, KW), lambda g, i: (0, 0)),
            pl.BlockSpec((S, KW), lambda g, i: (0, 0)),
        ],
        out_specs=pl.BlockSpec((BQ, GW), lambda g, i: (i, g)),
        out_shape=jax.ShapeDtypeStruct((S, QW), jnp.bfloat16),
        compiler_params=pltpu.CompilerParams(
            dimension_semantics=("parallel", "parallel")),
    )(q2d, k2d, v2d)

    hres, xf, logits = pl.pallas_call(
        _post_body,
        grid=(NB,),
        in_specs=[
            pl.BlockSpec((BT, QW), lambda i: (i, 0)),
            pl.BlockSpec((D, QW), lambda i: (0, 0)),
            pl.BlockSpec((BT, D), lambda i: (i, 0)),
            pl.BlockSpec((1, D), lambda i: (0, 0)),
            pl.BlockSpec((E, D), lambda i: (0, 0)),
        ],
        out_specs=[
            pl.BlockSpec((BT, D), lambda i: (i, 0)),
            pl.BlockSpec((BT, D), lambda i: (i, 0)),
            pl.BlockSpec((BT, E), lambda i: (i, 0)),
        ],
        out_shape=[
            jax.ShapeDtypeStruct((S, D), jnp.float32),
            jax.ShapeDtypeStruct((S, D), jnp.float32),
            jax.ShapeDtypeStruct((S, E), jnp.float32),
        ],
        compiler_params=pltpu.CompilerParams(
            dimension_semantics=("parallel",)),
    )(attn2d, wo2, x2d, plw, gate_w)

    w0, w1, pos, nbk = pl.pallas_call(
        _route_body,
        grid=(1,),
        in_specs=[pl.BlockSpec((S, E), lambda i: (0, 0))],
        out_specs=[
            pl.BlockSpec((S, 1), lambda i: (0, 0)),
            pl.BlockSpec((S, 1), lambda i: (0, 0)),
            pl.BlockSpec((NP, 1), lambda i: (0, 0)),
            pl.BlockSpec((1, E), lambda i: (0, 0)),
        ],
        out_shape=[
            jax.ShapeDtypeStruct((S, 1), jnp.float32),
            jax.ShapeDtypeStruct((S, 1), jnp.float32),
            jax.ShapeDtypeStruct((NP, 1), jnp.int32),
            jax.ShapeDtypeStruct((1, E), jnp.int32),
        ],
    )(logits)

    pos1d = pos.reshape(NP)
    cnb = jnp.cumsum(nbk[0])
    eid = jnp.minimum(
        jnp.sum(cnb[:, None] <= jnp.arange(NG)[None, :], axis=0),
        E - 1).astype(jnp.int32)

    mesh = plsc.VectorSubcoreMesh(core_axis_name="c", subcore_axis_name="s")

    xs = pl.kernel(
        _sc_dispatch_body,
        mesh=mesh,
        out_type=jax.ShapeDtypeStruct((P, D), jnp.float32),
        scratch_types=[
            pltpu.VMEM((64,), jnp.int32),
            pltpu.VMEM((64, D), jnp.float32),
            pltpu.SemaphoreType.DMA,
            pltpu.SemaphoreType.DMA,
        ],
    )(pos1d, xf)

    osorted = pl.pallas_call(
        _ffn_body,
        grid_spec=pltpu.PrefetchScalarGridSpec(
            num_scalar_prefetch=1,
            grid=(NG,),
            in_specs=[
                pl.BlockSpec((BTM, D), lambda i, eid_ref: (i, 0)),
                pl.BlockSpec((1, F, D), lambda i, eid_ref: (eid_ref[i], 0, 0)),
                pl.BlockSpec((1, F, D), lambda i, eid_ref: (eid_ref[i], 0, 0)),
                pl.BlockSpec((1, D, F), lambda i, eid_ref: (eid_ref[i], 0, 0)),
            ],
            out_specs=pl.BlockSpec((BTM, D), lambda i, eid_ref: (i, 0)),
        ),
        out_shape=jax.ShapeDtypeStruct((P, D), jnp.float32),
        compiler_params=pltpu.CompilerParams(
            dimension_semantics=("parallel",)),
    )(eid, xs, wg_b, wu_b, wd_b)

    a0, a1 = pl.kernel(
        _sc_combine_body,
        mesh=mesh,
        out_type=[
            jax.ShapeDtypeStruct((S, D), jnp.float32),
            jax.ShapeDtypeStruct((S, D), jnp.float32),
        ],
        scratch_types=[
            pltpu.VMEM((S // _SC_NW,), jnp.int32),
            pltpu.VMEM((S // _SC_NW, D), jnp.float32),
            pltpu.SemaphoreType.DMA,
            pltpu.SemaphoreType.DMA,
        ],
    )(pos1d, osorted)

    out = pl.pallas_call(
        _combine_body,
        grid=(NB,),
        in_specs=[
            pl.BlockSpec((BT, D), lambda i: (i, 0)),
            pl.BlockSpec((BT, D), lambda i: (i, 0)),
            pl.BlockSpec((BT, D), lambda i: (i, 0)),
            pl.BlockSpec((BT, 1), lambda i: (i, 0)),
            pl.BlockSpec((BT, 1), lambda i: (i, 0)),
        ],
        out_specs=pl.BlockSpec((BT, D), lambda i: (i, 0)),
        out_shape=jax.ShapeDtypeStruct((S, D), jnp.float32),
        compiler_params=pltpu.CompilerParams(
            dimension_semantics=("parallel",)),
    )(hres, a0, a1, w0, w1)

    return out.reshape(B, S, D)


# speedup vs baseline: 2.5777x; 1.0031x over previous
import functools

import jax
import jax.numpy as jnp
from jax import lax
from jax.experimental import pallas as pl
from jax.experimental.pallas import tpu as pltpu
from jax.experimental.pallas import tpu_sc as plsc

B, S, D = 1, 2048, 1024
H, KVH, HD = 16, 4, 64
E, TOPK, F = 8, 2, 768
EPS = 1e-06

BT = 512
NB = S // BT
BQ = 512

NP = TOPK * S
BTM = 128
P = NP + E * BTM
NG = P // BTM

_dot = functools.partial(jax.lax.dot_general, preferred_element_type=jnp.float32)

_SC_NC, _SC_NS = 2, 16
_SC_NW = _SC_NC * _SC_NS


def _qkv_body(h_ref, lnw_ref, wqkv_ref, cos_ref, sin_ref, qnw_ref, knw_ref,
              mgq_ref, rq_ref, tq_ref, mgk_ref, rk_ref, tk_ref,
              q_ref, k_ref, v_ref):
    h32 = h_ref[...]
    var = jnp.mean(h32 * h32, axis=1, keepdims=True)
    hn = (h32 * jax.lax.rsqrt(var + EPS)) * lnw_ref[...]
    qkv = _dot(hn.astype(jnp.bfloat16), wqkv_ref[...], (((1,), (1,)), ((), ())))
    cosb = cos_ref[...]
    sinb = sin_ref[...]

    def headnorm_rope(x, mg_ref, r_ref, w_ref, t_ref):
        xb = x.astype(jnp.bfloat16)
        msum = _dot(xb * xb, mg_ref[...], (((1,), (0,)), ((), ())))
        xn = (x * jax.lax.rsqrt(msum * (1.0 / HD) + EPS)) * w_ref[...]
        xr = _dot(xn.astype(jnp.bfloat16), r_ref[...], (((1,), (0,)), ((), ())))
        cos = _dot(cosb, t_ref[...], (((1,), (0,)), ((), ())))
        sin = _dot(sinb, t_ref[...], (((1,), (0,)), ((), ())))
        return (xn * cos + xr * sin).astype(jnp.bfloat16)

    q = qkv[:, :H * HD]
    k = qkv[:, H * HD:(H + KVH) * HD]
    v = qkv[:, (H + KVH) * HD:]
    q_ref[...] = headnorm_rope(q, mgq_ref, rq_ref, qnw_ref, tq_ref)
    k_ref[...] = headnorm_rope(k, mgk_ref, rk_ref, knw_ref, tk_ref)
    v_ref[...] = v.astype(jnp.bfloat16)


def _attn_body(q_ref, k_ref, v_ref, o_ref):
    g = pl.program_id(0)
    kp = k_ref[...]
    vp = v_ref[...]
    kk = kp[:, :HD]
    vv = vp[:, :HD]
    for j in range(1, KVH):
        kk = jnp.where(g == j, kp[:, j * HD:(j + 1) * HD], kk)
        vv = jnp.where(g == j, vp[:, j * HD:(j + 1) * HD], vv)
    outs = []
    for sub in range(H // KVH):
        qh = q_ref[:, sub * HD:(sub + 1) * HD]
        s = _dot(qh, kk, (((1,), (1,)), ((), ())))
        p = jnp.exp(s)
        l = jnp.sum(p, axis=1, keepdims=True)
        o = _dot(p.astype(jnp.bfloat16), vv, (((1,), (0,)), ((), ())))
        outs.append(o / l)
    o_ref[...] = jnp.concatenate(outs, axis=1).astype(jnp.bfloat16)


def _post_body(attn_ref, wo_ref, res_ref, plw_ref, gate_ref,
               h_ref, x_ref, lg_ref):
    acc = _dot(attn_ref[...], wo_ref[...], (((1,), (1,)), ((), ())))
    hout = res_ref[...] + acc
    var = jnp.mean(hout * hout, axis=1, keepdims=True)
    xn = (hout * jax.lax.rsqrt(var + EPS)) * plw_ref[...]
    h_ref[...] = hout
    x_ref[...] = xn
    lg_ref[...] = _dot(xn, gate_ref[...], (((1,), (1,)), ((), ())))


def _route_body(lg_ref, w0_ref, w1_ref, pos_ref, eid_ref):
    lg = lg_ref[...]
    m = jnp.max(lg, axis=1, keepdims=True)
    p = jnp.exp(lg - m)
    rw = p / jnp.sum(p, axis=1, keepdims=True)
    lanes = jax.lax.broadcasted_iota(jnp.int32, (S, E), 1)
    m0 = jnp.max(rw, axis=1, keepdims=True)
    i0 = jnp.min(jnp.where(rw == m0, lanes, E), axis=1, keepdims=True)
    sel0 = lanes == i0
    rw2 = jnp.where(sel0, -1.0, rw)
    m1 = jnp.max(rw2, axis=1, keepdims=True)
    i1 = jnp.min(jnp.where(rw2 == m1, lanes, E), axis=1, keepdims=True)
    sel1 = lanes == i1
    wsum = m0 + m1
    w0_ref[...] = m0 / wsum
    w1_ref[...] = m1 / wsum

    oh = jnp.concatenate([sel0.astype(jnp.float32), sel1.astype(jnp.float32)],
                         axis=0)
    CH = 512
    r_iota = jax.lax.broadcasted_iota(jnp.int32, (CH, CH), 0)
    c_iota = jax.lax.broadcasted_iota(jnp.int32, (CH, CH), 1)
    tri = (c_iota < r_iota).astype(jnp.float32)
    carry = jnp.zeros((1, E), jnp.float32)
    ranks = []
    for c in range(NP // CH):
        ohc = oh[c * CH:(c + 1) * CH]
        ranks.append(_dot(tri, ohc, (((1,), (0,)), ((), ()))) + carry)
        carry = carry + jnp.sum(ohc, axis=0, keepdims=True)
    rank = jnp.concatenate(ranks, axis=0)
    nbk = jnp.floor((carry + (BTM - 1)) * (1.0 / BTM))
    er_iota = jax.lax.broadcasted_iota(jnp.int32, (E, E), 0)
    ec_iota = jax.lax.broadcasted_iota(jnp.int32, (E, E), 1)
    tri_e = (er_iota < ec_iota).astype(jnp.float32)
    off = BTM * _dot(nbk, tri_e, (((1,), (0,)), ((), ())))
    pos_f = (jnp.sum(rank * oh, axis=1, keepdims=True)
             + jnp.sum(off * oh, axis=1, keepdims=True))
    pos_ref[...] = pos_f.astype(jnp.int32)
    i8 = (er_iota == ec_iota).astype(jnp.float32)
    l8 = (er_iota >= ec_iota).astype(jnp.float32)
    nbk_col = _dot(i8, nbk, (((1,), (1,)), ((), ())))
    cnb_col = _dot(l8, nbk_col, (((1,), (0,)), ((), ())))
    bi = jax.lax.broadcasted_iota(jnp.int32, (E, NG), 1).astype(jnp.float32)
    g_le = (cnb_col <= bi).astype(jnp.float32)
    ones_row = jnp.ones((1, E), jnp.float32)
    eid_f = _dot(ones_row, g_le, (((1,), (0,)), ((), ())))
    eid_ref[...] = jnp.minimum(eid_f, E - 1).astype(jnp.int32)


def _sc_dispatch_body(pos_hbm, xf_hbm, xs_hbm, idx_v, rows_v, sem, sem2):
    wid = lax.axis_index("s") * _SC_NC + lax.axis_index("c")
    pairs_per_w = NP // _SC_NW
    tok_shift = jnp.where(wid < _SC_NW // 2, 0, S)
    for s_ in range(pairs_per_w // 64):
        j0 = wid * pairs_per_w + s_ * 64
        cp1 = pltpu.async_copy(xf_hbm.at[pl.ds(j0 - tok_shift, 64)], rows_v,
                               sem)
        cp2 = pltpu.async_copy(pos_hbm.at[pl.ds(j0, 64)], idx_v, sem2)
        cp1.wait()
        cp2.wait()
        pltpu.async_copy(rows_v, xs_hbm.at[idx_v], sem).wait()


def _sc_combine_body(pos_hbm, of_hbm, a0_hbm, a1_hbm, idx_v, rows_v, sem,
                     sem2):
    wid = lax.axis_index("s") * _SC_NC + lax.axis_index("c")
    tok_per_w = S // _SC_NW
    off = wid * tok_per_w
    for part in range(2):
        out_hbm = a0_hbm if part == 0 else a1_hbm
        pltpu.async_copy(pos_hbm.at[pl.ds(part * S + off, tok_per_w)], idx_v,
                         sem2).wait()
        pltpu.async_copy(of_hbm.at[idx_v], rows_v, sem).wait()
        pltpu.async_copy(rows_v, out_hbm.at[pl.ds(off, tok_per_w)], sem).wait()


def _ffn_body(eid_ref, x_ref, wg_ref, wu_ref, wd_ref, o_ref):
    x = x_ref[...].astype(jnp.bfloat16)
    g = _dot(x, wg_ref[0], (((1,), (1,)), ((), ())))
    u = _dot(x, wu_ref[0], (((1,), (1,)), ((), ())))
    hexp = ((g * jax.nn.sigmoid(g)) * u).astype(jnp.bfloat16)
    o_ref[...] = _dot(hexp, wd_ref[0], (((1,), (1,)), ((), ())))


def _combine_body(res_ref, a0_ref, a1_ref, w0_ref, w1_ref, o_ref):
    o_ref[...] = (res_ref[...]
                  + w0_ref[...] * a0_ref[...]
                  + w1_ref[...] * a1_ref[...])


def kernel(hidden_states, start_pos, position_cos, position_sin, attention_mask,
           Wq, Wk, Wv, Wo, q_norm_w, k_norm_w, input_ln_w, post_ln_w,
           gate_w, Wg, Wu, Wd):
    x2d = hidden_states.reshape(S, D)
    wqkv = jnp.concatenate([Wq, Wk, Wv], axis=0).astype(jnp.bfloat16)
    wo2 = Wo.astype(jnp.bfloat16)
    wg_b = Wg.astype(jnp.bfloat16)
    wu_b = Wu.astype(jnp.bfloat16)
    wd_b = Wd.astype(jnp.bfloat16)
    lnw = input_ln_w.reshape(1, D)
    plw = post_ln_w.reshape(1, D)

    def rope_consts(nh):
        w = nh * HD
        jj = jnp.arange(w)[:, None]
        ll = jnp.arange(w)[None, :]
        g, p = ll // HD, ll % HD
        mg = (jj // HD == g).astype(jnp.bfloat16)
        r = (jnp.where((p < HD // 2) & (jj == g * HD + p + HD // 2), -1.0, 0.0)
             + jnp.where((p >= HD // 2) & (jj == g * HD + p - HD // 2),
                         1.0, 0.0)).astype(jnp.bfloat16)
        return mg, r

    mgq, rq = rope_consts(H)
    mgk, rk = rope_consts(KVH)
    jhd = jnp.arange(HD)[:, None]
    tq = ((jnp.arange(H * HD)[None, :] % HD == jhd)
          * (HD ** -0.5)).astype(jnp.bfloat16)
    tk = (jnp.arange(KVH * HD)[None, :] % HD == jhd).astype(jnp.bfloat16)
    cosb = position_cos.astype(jnp.bfloat16)
    sinb = position_sin.astype(jnp.bfloat16)
    qnw = jnp.tile(q_norm_w, H).reshape(1, H * HD)
    knw = jnp.tile(k_norm_w, KVH).reshape(1, KVH * HD)

    QW, KW = H * HD, KVH * HD
    q2d, k2d, v2d = pl.pallas_call(
        _qkv_body,
        grid=(NB,),
        in_specs=[
            pl.BlockSpec((BT, D), lambda i: (i, 0)),
            pl.BlockSpec((1, D), lambda i: (0, 0)),
            pl.BlockSpec(((H + 2 * KVH) * HD, D), lambda i: (0, 0)),
            pl.BlockSpec((BT, HD), lambda i: (i, 0)),
            pl.BlockSpec((BT, HD), lambda i: (i, 0)),
            pl.BlockSpec((1, QW), lambda i: (0, 0)),
            pl.BlockSpec((1, KW), lambda i: (0, 0)),
            pl.BlockSpec((QW, QW), lambda i: (0, 0)),
            pl.BlockSpec((QW, QW), lambda i: (0, 0)),
            pl.BlockSpec((HD, QW), lambda i: (0, 0)),
            pl.BlockSpec((KW, KW), lambda i: (0, 0)),
            pl.BlockSpec((KW, KW), lambda i: (0, 0)),
            pl.BlockSpec((HD, KW), lambda i: (0, 0)),
        ],
        out_specs=[
            pl.BlockSpec((BT, QW), lambda i: (i, 0)),
            pl.BlockSpec((BT, KW), lambda i: (i, 0)),
            pl.BlockSpec((BT, KW), lambda i: (i, 0)),
        ],
        out_shape=[
            jax.ShapeDtypeStruct((S, QW), jnp.bfloat16),
            jax.ShapeDtypeStruct((S, KW), jnp.bfloat16),
            jax.ShapeDtypeStruct((S, KW), jnp.bfloat16),
        ],
        compiler_params=pltpu.CompilerParams(
            dimension_semantics=("parallel",)),
    )(x2d, lnw, wqkv, cosb, sinb, qnw, knw, mgq, rq, tq, mgk, rk, tk)

    GW = (H // KVH) * HD
    attn2d = pl.pallas_call(
        _attn_body,
        grid=(KVH, S // BQ),
        in_specs=[
            pl.BlockSpec((BQ, GW), lambda g, i: (i, g)),
            pl.BlockSpec((S, KW), lambda g, i: (0, 0)),
            pl.BlockSpec((S, KW), lambda g, i: (0, 0)),
        ],
        out_specs=pl.BlockSpec((BQ, GW), lambda g, i: (i, g)),
        out_shape=jax.ShapeDtypeStruct((S, QW), jnp.bfloat16),
        compiler_params=pltpu.CompilerParams(
            dimension_semantics=("parallel", "parallel")),
    )(q2d, k2d, v2d)

    hres, xf, logits = pl.pallas_call(
        _post_body,
        grid=(NB,),
        in_specs=[
            pl.BlockSpec((BT, QW), lambda i: (i, 0)),
            pl.BlockSpec((D, QW), lambda i: (0, 0)),
            pl.BlockSpec((BT, D), lambda i: (i, 0)),
            pl.BlockSpec((1, D), lambda i: (0, 0)),
            pl.BlockSpec((E, D), lambda i: (0, 0)),
        ],
        out_specs=[
            pl.BlockSpec((BT, D), lambda i: (i, 0)),
            pl.BlockSpec((BT, D), lambda i: (i, 0)),
            pl.BlockSpec((BT, E), lambda i: (i, 0)),
        ],
        out_shape=[
            jax.ShapeDtypeStruct((S, D), jnp.float32),
            jax.ShapeDtypeStruct((S, D), jnp.float32),
            jax.ShapeDtypeStruct((S, E), jnp.float32),
        ],
        compiler_params=pltpu.CompilerParams(
            dimension_semantics=("parallel",)),
    )(attn2d, wo2, x2d, plw, gate_w)

    w0, w1, pos, eid2 = pl.pallas_call(
        _route_body,
        grid=(1,),
        in_specs=[pl.BlockSpec((S, E), lambda i: (0, 0))],
        out_specs=[
            pl.BlockSpec((S, 1), lambda i: (0, 0)),
            pl.BlockSpec((S, 1), lambda i: (0, 0)),
            pl.BlockSpec((NP, 1), lambda i: (0, 0)),
            pl.BlockSpec((1, NG), lambda i: (0, 0)),
        ],
        out_shape=[
            jax.ShapeDtypeStruct((S, 1), jnp.float32),
            jax.ShapeDtypeStruct((S, 1), jnp.float32),
            jax.ShapeDtypeStruct((NP, 1), jnp.int32),
            jax.ShapeDtypeStruct((1, NG), jnp.int32),
        ],
    )(logits)

    pos1d = pos.reshape(NP)
    eid = eid2.reshape(NG)

    mesh = plsc.VectorSubcoreMesh(core_axis_name="c", subcore_axis_name="s")

    xs = pl.kernel(
        _sc_dispatch_body,
        mesh=mesh,
        out_type=jax.ShapeDtypeStruct((P, D), jnp.float32),
        scratch_types=[
            pltpu.VMEM((64,), jnp.int32),
            pltpu.VMEM((64, D), jnp.float32),
            pltpu.SemaphoreType.DMA,
            pltpu.SemaphoreType.DMA,
        ],
    )(pos1d, xf)

    osorted = pl.pallas_call(
        _ffn_body,
        grid_spec=pltpu.PrefetchScalarGridSpec(
            num_scalar_prefetch=1,
            grid=(NG,),
            in_specs=[
                pl.BlockSpec((BTM, D), lambda i, eid_ref: (i, 0)),
                pl.BlockSpec((1, F, D), lambda i, eid_ref: (eid_ref[i], 0, 0)),
                pl.BlockSpec((1, F, D), lambda i, eid_ref: (eid_ref[i], 0, 0)),
                pl.BlockSpec((1, D, F), lambda i, eid_ref: (eid_ref[i], 0, 0)),
            ],
            out_specs=pl.BlockSpec((BTM, D), lambda i, eid_ref: (i, 0)),
        ),
        out_shape=jax.ShapeDtypeStruct((P, D), jnp.float32),
        compiler_params=pltpu.CompilerParams(
            dimension_semantics=("parallel",)),
    )(eid, xs, wg_b, wu_b, wd_b)

    a0, a1 = pl.kernel(
        _sc_combine_body,
        mesh=mesh,
        out_type=[
            jax.ShapeDtypeStruct((S, D), jnp.float32),
            jax.ShapeDtypeStruct((S, D), jnp.float32),
        ],
        scratch_types=[
            pltpu.VMEM((S // _SC_NW,), jnp.int32),
            pltpu.VMEM((S // _SC_NW, D), jnp.float32),
            pltpu.SemaphoreType.DMA,
            pltpu.SemaphoreType.DMA,
        ],
    )(pos1d, osorted)

    out = pl.pallas_call(
        _combine_body,
        grid=(NB,),
        in_specs=[
            pl.BlockSpec((BT, D), lambda i: (i, 0)),
            pl.BlockSpec((BT, D), lambda i: (i, 0)),
            pl.BlockSpec((BT, D), lambda i: (i, 0)),
            pl.BlockSpec((BT, 1), lambda i: (i, 0)),
            pl.BlockSpec((BT, 1), lambda i: (i, 0)),
        ],
        out_specs=pl.BlockSpec((BT, D), lambda i: (i, 0)),
        out_shape=jax.ShapeDtypeStruct((S, D), jnp.float32),
        compiler_params=pltpu.CompilerParams(
            dimension_semantics=("parallel",)),
    )(hres, a0, a1, w0, w1)

    return out.reshape(B, S, D)
